# Initial kernel scaffold; baseline (speedup 1.0000x reference)
import jax
import jax.numpy as jnp
from jax.experimental import pallas as pl


def kernel(x, edge_index, W_gl, a_gl, W0, b0, W1, b1):
    raise NotImplementedError("write your pallas kernel here")


# speedup vs baseline: 5.9757x; 5.9757x over previous
import functools

import jax
import jax.numpy as jnp
from jax import lax
from jax.experimental import pallas as pl
from jax.experimental.pallas import tpu as pltpu
from jax.experimental.pallas import tpu_sc as plsc

N = 10000
E = 320000
D = 128
HG = 32
H = 128
C_OUT = 40
CP = 48
ZW = 144
LAMB1 = 0.01
LAMB2 = 0.0001

NC = 2
NS = 16
NW = NC * NS
EPW = E // NW
CHUNK = 80
NCH = EPW // CHUNK
NP = 10240
RPT = NP // NS
L = 16

_mesh = plsc.VectorSubcoreMesh(core_axis_name="c", subcore_axis_name="s")
_sc_params = pltpu.CompilerParams(use_tc_tiling_on_sc=False,
                                  needs_layout_passes=False)


def _tc1_body(x_ref, wgl_ref, w0_ref, h_ref, p_ref):
    xb = x_ref[...]
    h_ref[...] = jnp.dot(xb, wgl_ref[...], preferred_element_type=jnp.float32)
    p_ref[...] = jnp.dot(xb, w0_ref[...], preferred_element_type=jnp.float32)


def _tc1(x, W_gl, W0):
    BR = 1000
    return pl.pallas_call(
        _tc1_body,
        grid=(N // BR,),
        in_specs=[pl.BlockSpec((BR, D), lambda i: (i, 0)),
                  pl.BlockSpec((D, HG), lambda i: (0, 0)),
                  pl.BlockSpec((D, H), lambda i: (0, 0))],
        out_specs=[pl.BlockSpec((BR, HG), lambda i: (i, 0)),
                   pl.BlockSpec((BR, H), lambda i: (i, 0))],
        out_shape=[jax.ShapeDtypeStruct((N, HG), jnp.float32),
                   jax.ShapeDtypeStruct((N, H), jnp.float32)],
    )(x, W_gl, W0)


@functools.partial(
    pl.kernel,
    out_type=[
        jax.ShapeDtypeStruct((NC * NP, ZW), jnp.float32),
        jax.ShapeDtypeStruct((E,), jnp.float32),
        jax.ShapeDtypeStruct((E,), jnp.float32),
    ],
    mesh=_mesh,
    compiler_params=_sc_params,
    scratch_types=[
        pltpu.VMEM((HG,), jnp.float32),
        pltpu.VMEM((CHUNK,), jnp.int32),
        pltpu.VMEM((CHUNK,), jnp.int32),
        pltpu.VMEM((CHUNK, HG), jnp.float32),
        pltpu.VMEM((CHUNK, HG), jnp.float32),
        pltpu.VMEM((CHUNK, D), jnp.float32),
        pltpu.VMEM((CHUNK, ZW), jnp.float32),
        pltpu.VMEM((CHUNK,), jnp.float32),
        pltpu.VMEM((CHUNK,), jnp.float32),
        pltpu.VMEM_SHARED((NP, ZW), jnp.float32),
        pltpu.SemaphoreType.DMA,
    ],
)
def _sc_pass1(h_hbm, p_hbm, src_hbm, dst_hbm, a_hbm, zero_hbm,
              zacc_hbm, ex_hbm, sq_hbm,
              a_v, idxs_v, idxd_v, hs_v, hd_v, ps_v, pw_v, exb_v, sqb_v,
              zsh, sem):
    c = lax.axis_index("c")
    s = lax.axis_index("s")
    wid = s * NC + c
    r0 = s * RPT
    pltpu.sync_copy(zero_hbm.at[pl.ds(r0, RPT)], zsh.at[pl.ds(r0, RPT)])
    pltpu.sync_copy(a_hbm, a_v)
    plsc.subcore_barrier()

    iota = lax.iota(jnp.int32, L)
    av = [a_v[pl.ds(kk * L, L)] for kk in range(HG // L)]

    def chunk_body(ch, _):
        base = wid * EPW + ch * CHUNK
        pltpu.sync_copy(src_hbm.at[pl.ds(base, CHUNK)], idxs_v)
        pltpu.sync_copy(dst_hbm.at[pl.ds(base, CHUNK)], idxd_v)
        cp1 = pltpu.async_copy(h_hbm.at[idxs_v], hs_v, sem)
        cp2 = pltpu.async_copy(h_hbm.at[idxd_v], hd_v, sem)
        cp3 = pltpu.async_copy(p_hbm.at[idxs_v], ps_v, sem)
        cp1.wait()
        cp2.wait()
        cp3.wait()

        def group_body(g, carry):
            rows = g * L + iota
            e_acc = jnp.zeros((L,), jnp.float32)
            q_acc = jnp.zeros((L,), jnp.float32)
            for k in range(HG):
                colk = jnp.full((L,), k, jnp.int32)
                dk = (plsc.load_gather(hs_v, [rows, colk])
                      - plsc.load_gather(hd_v, [rows, colk]))
                e_acc = e_acc + av[k // L][k % L] * jnp.abs(dk)
                q_acc = q_acc + dk * dk
            ex = jnp.exp(jnp.maximum(e_acc, 0.0))
            exb_v[pl.ds(g * L, L)] = ex
            sqb_v[pl.ds(g * L, L)] = q_acc
            for r in range(L):
                row = g * L + r
                exi = ex[r]
                for j in range(D // L):
                    pw_v[row, pl.ds(j * L, L)] = ps_v[row, pl.ds(j * L, L)] * exi
                pw_v[row, pl.ds(D, L)] = jnp.where(iota == 0, exi, 0.0)
            return carry

        lax.fori_loop(0, CHUNK // L, group_body, 0)

        pltpu.sync_copy(pw_v, zsh.at[idxd_v], add=True)
        pltpu.sync_copy(exb_v, ex_hbm.at[pl.ds(base, CHUNK)])
        pltpu.sync_copy(sqb_v, sq_hbm.at[pl.ds(base, CHUNK)])
        return _

    lax.fori_loop(0, NCH, chunk_body, 0)
    plsc.subcore_barrier()
    pltpu.sync_copy(zsh.at[pl.ds(r0, RPT)],
                    zacc_hbm.at[pl.ds(c * NP + r0, RPT)])


def _tc2_body(zacc_ref, b0_ref, w1_ref, q_ref, sinv_ref):
    za = zacc_ref[0] + zacc_ref[1]
    sden = za[:, D:D + 1]
    sinv = 1.0 / (sden + 1e-16)
    z = jnp.maximum(za[:, :D] * sinv + b0_ref[...], 0.0)
    q_ref[...] = jnp.dot(z, w1_ref[...], preferred_element_type=jnp.float32)
    sinv_ref[...] = sinv


def _tc2(zacc3, b0r, W1p):
    BR = 1000
    return pl.pallas_call(
        _tc2_body,
        grid=(N // BR,),
        in_specs=[pl.BlockSpec((2, BR, ZW), lambda i: (0, i, 0)),
                  pl.BlockSpec((1, D), lambda i: (0, 0)),
                  pl.BlockSpec((D, CP), lambda i: (0, 0))],
        out_specs=[pl.BlockSpec((BR, CP), lambda i: (i, 0)),
                   pl.BlockSpec((BR, 1), lambda i: (i, 0))],
        out_shape=[jax.ShapeDtypeStruct((N, CP), jnp.float32),
                   jax.ShapeDtypeStruct((N, 1), jnp.float32)],
    )(zacc3, b0r, W1p)


@functools.partial(
    pl.kernel,
    out_type=[
        jax.ShapeDtypeStruct((E,), jnp.float32),
        jax.ShapeDtypeStruct((NC * NP, CP), jnp.float32),
        jax.ShapeDtypeStruct((2 * NW, L), jnp.float32),
    ],
    mesh=_mesh,
    compiler_params=_sc_params,
    scratch_types=[
        pltpu.VMEM((N,), jnp.float32),
        pltpu.VMEM((CHUNK,), jnp.int32),
        pltpu.VMEM((CHUNK,), jnp.int32),
        pltpu.VMEM((CHUNK,), jnp.float32),
        pltpu.VMEM((CHUNK,), jnp.float32),
        pltpu.VMEM((CHUNK,), jnp.float32),
        pltpu.VMEM((CHUNK, CP), jnp.float32),
        pltpu.VMEM((CHUNK, CP), jnp.float32),
        pltpu.VMEM((2, L), jnp.float32),
        pltpu.VMEM_SHARED((NP, CP), jnp.float32),
        pltpu.SemaphoreType.DMA,
    ],
)
def _sc_pass3(q_hbm, sinv_hbm, src_hbm, dst_hbm, ex_hbm, sq_hbm, zero_hbm,
              w_hbm, oacc_hbm, lp_hbm,
              sinv_v, idxs_v, idxd_v, exb_v, sqb_v, wb_v, qs_v, qw_v, lbuf,
              osh, sem):
    c = lax.axis_index("c")
    s = lax.axis_index("s")
    wid = s * NC + c
    r0 = s * RPT
    pltpu.sync_copy(zero_hbm.at[pl.ds(r0, RPT)], osh.at[pl.ds(r0, RPT)])
    pltpu.sync_copy(sinv_hbm, sinv_v)
    plsc.subcore_barrier()

    def chunk_body(ch, carry):
        l1, l2 = carry
        base = wid * EPW + ch * CHUNK
        pltpu.sync_copy(src_hbm.at[pl.ds(base, CHUNK)], idxs_v)
        pltpu.sync_copy(dst_hbm.at[pl.ds(base, CHUNK)], idxd_v)
        pltpu.sync_copy(ex_hbm.at[pl.ds(base, CHUNK)], exb_v)
        pltpu.sync_copy(sq_hbm.at[pl.ds(base, CHUNK)], sqb_v)
        pltpu.async_copy(q_hbm.at[idxs_v], qs_v, sem).wait()

        def group_body(g, car):
            a1, a2 = car
            sl = pl.ds(g * L, L)
            dvec = idxd_v[sl]
            sv = plsc.load_gather(sinv_v, [dvec])
            wv = exb_v[sl] * sv
            wb_v[sl] = wv
            for r in range(L):
                row = g * L + r
                wi = wv[r]
                for j in range(CP // L):
                    qw_v[row, pl.ds(j * L, L)] = qs_v[row, pl.ds(j * L, L)] * wi
            return (a1 + wv * sqb_v[sl], a2 + wv * wv)

        l1, l2 = lax.fori_loop(0, CHUNK // L, group_body, (l1, l2))

        pltpu.sync_copy(qw_v, osh.at[idxd_v], add=True)
        pltpu.sync_copy(wb_v, w_hbm.at[pl.ds(base, CHUNK)])
        return (l1, l2)

    zl = jnp.zeros((L,), jnp.float32)
    l1, l2 = lax.fori_loop(0, NCH, chunk_body, (zl, zl))
    lbuf[0, :] = l1
    lbuf[1, :] = l2
    plsc.subcore_barrier()
    pltpu.sync_copy(osh.at[pl.ds(r0, RPT)],
                    oacc_hbm.at[pl.ds(c * NP + r0, RPT)])
    pltpu.sync_copy(lbuf, lp_hbm.at[pl.ds(wid * 2, 2)])


def _tc4_body(oacc_ref, b1_ref, lp_ref, z_ref, loss_ref):
    z_ref[...] = oacc_ref[0] + oacc_ref[1] + b1_ref[...]

    @pl.when(pl.program_id(0) == 0)
    def _():
        lp = lp_ref[...]
        rid = lax.broadcasted_iota(jnp.int32, lp.shape, 0)
        l1 = jnp.sum(jnp.where(rid % 2 == 0, lp, 0.0))
        l2 = jnp.sum(jnp.where(rid % 2 == 1, lp, 0.0))
        loss_ref[...] = jnp.reshape(LAMB1 * 0.5 * l1 + LAMB2 * l2, (1, 1))


def _tc4(oacc3, b1r, lp):
    BR = 1000
    return pl.pallas_call(
        _tc4_body,
        grid=(N // BR,),
        in_specs=[pl.BlockSpec((2, BR, CP), lambda i: (0, i, 0)),
                  pl.BlockSpec((1, CP), lambda i: (0, 0)),
                  pl.BlockSpec((2 * NW, L), lambda i: (0, 0))],
        out_specs=[pl.BlockSpec((BR, CP), lambda i: (i, 0)),
                   pl.BlockSpec((1, 1), lambda i: (0, 0))],
        out_shape=[jax.ShapeDtypeStruct((N, CP), jnp.float32),
                   jax.ShapeDtypeStruct((1, 1), jnp.float32)],
    )(oacc3, b1r, lp)


def kernel(x, edge_index, W_gl, a_gl, W0, b0, W1, b1):
    src = edge_index[0]
    dst = edge_index[1]
    h, p = _tc1(x, W_gl, W0)

    zeros_z = jnp.zeros((NP, ZW), jnp.float32)
    zacc, ex, sq = _sc_pass1(h, p, src, dst, a_gl, zeros_z)

    W1p = jnp.pad(W1, ((0, 0), (0, CP - C_OUT)))
    q, sinv = _tc2(zacc.reshape(NC, NP, ZW), b0.reshape(1, D), W1p)

    zeros_o = jnp.zeros((NP, CP), jnp.float32)
    w, oacc, lp = _sc_pass3(q, sinv.reshape(N), src, dst, ex, sq, zeros_o)

    b1p = jnp.pad(b1, (0, CP - C_OUT)).reshape(1, CP)
    zout, loss = _tc4(oacc.reshape(NC, NP, CP), b1p, lp)
    return zout[:, :C_OUT], w, loss[0, 0]


# speedup vs baseline: 6.9773x; 1.1676x over previous
import functools

import jax
import jax.numpy as jnp
from jax import lax
from jax.experimental import pallas as pl
from jax.experimental.pallas import tpu as pltpu
from jax.experimental.pallas import tpu_sc as plsc

N = 10000
E = 320000
D = 128
HG = 32
H = 128
C_OUT = 40
CP = 48
ZW = 144
LAMB1 = 0.01
LAMB2 = 0.0001

NC = 2
NS = 16
NW = NC * NS
EPW = E // NW
CHUNK = 80
NCH = EPW // CHUNK
NP = 10112
RPT = NP // NS
L = 16

_mesh = plsc.VectorSubcoreMesh(core_axis_name="c", subcore_axis_name="s")
_sc_params = pltpu.CompilerParams(use_tc_tiling_on_sc=False,
                                  needs_layout_passes=False)


def _tc1_body(x_ref, wgl_ref, w0_ref, h_ref, p_ref):
    xb = x_ref[...]
    h_ref[...] = jnp.dot(xb, wgl_ref[...], preferred_element_type=jnp.float32)
    p_ref[...] = jnp.dot(xb, w0_ref[...], preferred_element_type=jnp.float32)


def _tc1(x, W_gl, W0):
    BR = 1000
    return pl.pallas_call(
        _tc1_body,
        grid=(N // BR,),
        in_specs=[pl.BlockSpec((BR, D), lambda i: (i, 0)),
                  pl.BlockSpec((D, HG), lambda i: (0, 0)),
                  pl.BlockSpec((D, H), lambda i: (0, 0))],
        out_specs=[pl.BlockSpec((BR, HG), lambda i: (i, 0)),
                   pl.BlockSpec((BR, H), lambda i: (i, 0))],
        out_shape=[jax.ShapeDtypeStruct((N, HG), jnp.float32),
                   jax.ShapeDtypeStruct((N, H), jnp.float32)],
    )(x, W_gl, W0)


@functools.partial(
    pl.kernel,
    out_type=[
        jax.ShapeDtypeStruct((NC * NP, ZW), jnp.float32),
        jax.ShapeDtypeStruct((E + CHUNK,), jnp.float32),
        jax.ShapeDtypeStruct((E + CHUNK,), jnp.float32),
    ],
    mesh=_mesh,
    compiler_params=_sc_params,
    scratch_types=[
        pltpu.VMEM((HG,), jnp.float32),
        [pltpu.VMEM((CHUNK,), jnp.int32)] * 2,
        [pltpu.VMEM((CHUNK,), jnp.int32)] * 2,
        [pltpu.VMEM((CHUNK, HG), jnp.float32)] * 2,
        [pltpu.VMEM((CHUNK, HG), jnp.float32)] * 2,
        pltpu.VMEM((CHUNK, D), jnp.float32),
        pltpu.VMEM((CHUNK, ZW), jnp.float32),
        pltpu.VMEM((CHUNK,), jnp.float32),
        pltpu.VMEM((CHUNK,), jnp.float32),
        pltpu.VMEM_SHARED((NP, ZW), jnp.float32),
        [pltpu.SemaphoreType.DMA] * 2,
        pltpu.SemaphoreType.DMA,
    ],
)
def _sc_pass1(h_hbm, p_hbm, src_hbm, dst_hbm, a_hbm, zero_hbm,
              zacc_hbm, ex_hbm, sq_hbm,
              a_v, idxs_v, idxd_v, hs_v, hd_v, ps_v, pw_v, exb_v, sqb_v,
              zsh, sem, sem_ps):
    c = lax.axis_index("c")
    s = lax.axis_index("s")
    wid = s * NC + c
    r0 = s * RPT
    pltpu.sync_copy(zero_hbm.at[pl.ds(r0, RPT)], zsh.at[pl.ds(r0, RPT)])
    pltpu.sync_copy(a_hbm, a_v)
    plsc.subcore_barrier()

    iota = lax.iota(jnp.int32, L)
    av = [a_v[pl.ds(kk * L, L)] for kk in range(HG // L)]
    base0 = wid * EPW

    def prefetch(ch, b):
        base = base0 + ch * CHUNK
        pltpu.sync_copy(src_hbm.at[pl.ds(base, CHUNK)], idxs_v[b])
        pltpu.sync_copy(dst_hbm.at[pl.ds(base, CHUNK)], idxd_v[b])
        pltpu.async_copy(h_hbm.at[idxs_v[b]], hs_v[b], sem[b])
        pltpu.async_copy(h_hbm.at[idxd_v[b]], hd_v[b], sem[b])

    def wait_gathers(b):
        pltpu.make_async_copy(h_hbm.at[idxs_v[b]], hs_v[b], sem[b]).wait()
        pltpu.make_async_copy(h_hbm.at[idxd_v[b]], hd_v[b], sem[b]).wait()

    prefetch(0, 0)

    def do_chunk(ch, b):
        base = base0 + ch * CHUNK
        wait_gathers(b)
        cp_ps = pltpu.async_copy(p_hbm.at[idxs_v[b]], ps_v, sem_ps)
        prefetch(ch + 1, 1 - b)

        def group_body(g, carry):
            rows = g * L + iota
            e_acc = jnp.zeros((L,), jnp.float32)
            q_acc = jnp.zeros((L,), jnp.float32)
            for k in range(HG):
                colk = jnp.full((L,), k, jnp.int32)
                dk = (plsc.load_gather(hs_v[b], [rows, colk])
                      - plsc.load_gather(hd_v[b], [rows, colk]))
                e_acc = e_acc + av[k // L][k % L] * jnp.abs(dk)
                q_acc = q_acc + dk * dk
            ex = jnp.exp(jnp.maximum(e_acc, 0.0))
            exb_v[pl.ds(g * L, L)] = ex
            sqb_v[pl.ds(g * L, L)] = q_acc
            return carry

        lax.fori_loop(0, CHUNK // L, group_body, 0)
        cp_ps.wait()

        def weight_body(g, carry):
            ex = exb_v[pl.ds(g * L, L)]
            for r in range(L):
                row = g * L + r
                exi = ex[r]
                for j in range(D // L):
                    pw_v[row, pl.ds(j * L, L)] = ps_v[row, pl.ds(j * L, L)] * exi
                pw_v[row, pl.ds(D, L)] = jnp.where(iota == 0, exi, 0.0)
            return carry

        lax.fori_loop(0, CHUNK // L, weight_body, 0)

        pltpu.sync_copy(pw_v, zsh.at[idxd_v[b]], add=True)
        pltpu.sync_copy(exb_v, ex_hbm.at[pl.ds(base, CHUNK)])
        pltpu.sync_copy(sqb_v, sq_hbm.at[pl.ds(base, CHUNK)])

    def pair_body(i, carry):
        do_chunk(2 * i, 0)
        do_chunk(2 * i + 1, 1)
        return carry

    lax.fori_loop(0, NCH // 2, pair_body, 0)
    do_chunk(NCH - 1, 0)
    wait_gathers(1)
    plsc.subcore_barrier()
    pltpu.sync_copy(zsh.at[pl.ds(r0, RPT)],
                    zacc_hbm.at[pl.ds(c * NP + r0, RPT)])


def _tc2_body(zacc_ref, b0_ref, w1_ref, q_ref, sinv_ref):
    za = zacc_ref[0] + zacc_ref[1]
    sden = za[:, D:D + 1]
    sinv = 1.0 / (sden + 1e-16)
    z = jnp.maximum(za[:, :D] * sinv + b0_ref[...], 0.0)
    q_ref[...] = jnp.dot(z, w1_ref[...], preferred_element_type=jnp.float32)
    sinv_ref[...] = sinv


def _tc2(zacc3, b0r, W1p):
    BR = 1000
    return pl.pallas_call(
        _tc2_body,
        grid=(N // BR,),
        in_specs=[pl.BlockSpec((2, BR, ZW), lambda i: (0, i, 0)),
                  pl.BlockSpec((1, D), lambda i: (0, 0)),
                  pl.BlockSpec((D, CP), lambda i: (0, 0))],
        out_specs=[pl.BlockSpec((BR, CP), lambda i: (i, 0)),
                   pl.BlockSpec((BR, 1), lambda i: (i, 0))],
        out_shape=[jax.ShapeDtypeStruct((N, CP), jnp.float32),
                   jax.ShapeDtypeStruct((N, 1), jnp.float32)],
    )(zacc3, b0r, W1p)


@functools.partial(
    pl.kernel,
    out_type=[
        jax.ShapeDtypeStruct((E,), jnp.float32),
        jax.ShapeDtypeStruct((NC * NP, CP), jnp.float32),
        jax.ShapeDtypeStruct((2 * NW, L), jnp.float32),
    ],
    mesh=_mesh,
    compiler_params=_sc_params,
    scratch_types=[
        pltpu.VMEM((N,), jnp.float32),
        [pltpu.VMEM((CHUNK,), jnp.int32)] * 2,
        [pltpu.VMEM((CHUNK,), jnp.int32)] * 2,
        [pltpu.VMEM((CHUNK,), jnp.float32)] * 2,
        [pltpu.VMEM((CHUNK,), jnp.float32)] * 2,
        pltpu.VMEM((CHUNK,), jnp.float32),
        [pltpu.VMEM((CHUNK, CP), jnp.float32)] * 2,
        pltpu.VMEM((CHUNK, CP), jnp.float32),
        pltpu.VMEM((2, L), jnp.float32),
        pltpu.VMEM_SHARED((NP, CP), jnp.float32),
        [pltpu.SemaphoreType.DMA] * 2,
    ],
)
def _sc_pass3(q_hbm, sinv_hbm, src_hbm, dst_hbm, ex_hbm, sq_hbm, zero_hbm,
              w_hbm, oacc_hbm, lp_hbm,
              sinv_v, idxs_v, idxd_v, exb_v, sqb_v, wb_v, qs_v, qw_v, lbuf,
              osh, sem):
    c = lax.axis_index("c")
    s = lax.axis_index("s")
    wid = s * NC + c
    r0 = s * RPT
    pltpu.sync_copy(zero_hbm.at[pl.ds(r0, RPT)], osh.at[pl.ds(r0, RPT)])
    pltpu.sync_copy(sinv_hbm, sinv_v)
    plsc.subcore_barrier()
    base0 = wid * EPW

    def prefetch(ch, b):
        base = base0 + ch * CHUNK
        pltpu.sync_copy(src_hbm.at[pl.ds(base, CHUNK)], idxs_v[b])
        pltpu.sync_copy(dst_hbm.at[pl.ds(base, CHUNK)], idxd_v[b])
        pltpu.sync_copy(ex_hbm.at[pl.ds(base, CHUNK)], exb_v[b])
        pltpu.sync_copy(sq_hbm.at[pl.ds(base, CHUNK)], sqb_v[b])
        pltpu.async_copy(q_hbm.at[idxs_v[b]], qs_v[b], sem[b])

    def wait_gathers(b):
        pltpu.make_async_copy(q_hbm.at[idxs_v[b]], qs_v[b], sem[b]).wait()

    prefetch(0, 0)

    def do_chunk(ch, b, carry):
        l1, l2 = carry
        base = base0 + ch * CHUNK
        wait_gathers(b)
        prefetch(ch + 1, 1 - b)

        def group_body(g, car):
            a1, a2 = car
            sl = pl.ds(g * L, L)
            dvec = idxd_v[b][sl]
            sv = plsc.load_gather(sinv_v, [dvec])
            wv = exb_v[b][sl] * sv
            wb_v[sl] = wv
            for r in range(L):
                row = g * L + r
                wi = wv[r]
                for j in range(CP // L):
                    qw_v[row, pl.ds(j * L, L)] = qs_v[b][row, pl.ds(j * L, L)] * wi
            return (a1 + wv * sqb_v[b][sl], a2 + wv * wv)

        l1, l2 = lax.fori_loop(0, CHUNK // L, group_body, (l1, l2))

        pltpu.sync_copy(qw_v, osh.at[idxd_v[b]], add=True)
        pltpu.sync_copy(wb_v, w_hbm.at[pl.ds(base, CHUNK)])
        return (l1, l2)

    def pair_body(i, carry):
        carry = do_chunk(2 * i, 0, carry)
        carry = do_chunk(2 * i + 1, 1, carry)
        return carry

    zl = jnp.zeros((L,), jnp.float32)
    l1, l2 = lax.fori_loop(0, NCH // 2, pair_body, (zl, zl))
    l1, l2 = do_chunk(NCH - 1, 0, (l1, l2))
    wait_gathers(1)
    lbuf[0, :] = l1
    lbuf[1, :] = l2
    plsc.subcore_barrier()
    pltpu.sync_copy(osh.at[pl.ds(r0, RPT)],
                    oacc_hbm.at[pl.ds(c * NP + r0, RPT)])
    pltpu.sync_copy(lbuf, lp_hbm.at[pl.ds(wid * 2, 2)])


def _tc4_body(oacc_ref, b1_ref, lp_ref, z_ref, loss_ref):
    z_ref[...] = oacc_ref[0] + oacc_ref[1] + b1_ref[...]

    @pl.when(pl.program_id(0) == 0)
    def _():
        lp = lp_ref[...]
        rid = lax.broadcasted_iota(jnp.int32, lp.shape, 0)
        l1 = jnp.sum(jnp.where(rid % 2 == 0, lp, 0.0))
        l2 = jnp.sum(jnp.where(rid % 2 == 1, lp, 0.0))
        loss_ref[...] = jnp.reshape(LAMB1 * 0.5 * l1 + LAMB2 * l2, (1, 1))


def _tc4(oacc3, b1r, lp):
    BR = 1000
    return pl.pallas_call(
        _tc4_body,
        grid=(N // BR,),
        in_specs=[pl.BlockSpec((2, BR, CP), lambda i: (0, i, 0)),
                  pl.BlockSpec((1, CP), lambda i: (0, 0)),
                  pl.BlockSpec((2 * NW, L), lambda i: (0, 0))],
        out_specs=[pl.BlockSpec((BR, CP), lambda i: (i, 0)),
                   pl.BlockSpec((1, 1), lambda i: (0, 0))],
        out_shape=[jax.ShapeDtypeStruct((N, CP), jnp.float32),
                   jax.ShapeDtypeStruct((1, 1), jnp.float32)],
    )(oacc3, b1r, lp)


def kernel(x, edge_index, W_gl, a_gl, W0, b0, W1, b1):
    pad = jnp.zeros((2, CHUNK), jnp.int32)
    eip = jnp.concatenate([edge_index, pad], axis=1)
    src = eip[0]
    dst = eip[1]
    h, p = _tc1(x, W_gl, W0)

    zeros_z = jnp.zeros((NP, ZW), jnp.float32)
    zacc, ex, sq = _sc_pass1(h, p, src, dst, a_gl, zeros_z)

    W1p = jnp.pad(W1, ((0, 0), (0, CP - C_OUT)))
    q, sinv = _tc2(zacc.reshape(NC, NP, ZW), b0.reshape(1, D), W1p)

    zeros_o = jnp.zeros((NP, CP), jnp.float32)
    w, oacc, lp = _sc_pass3(q, sinv.reshape(N), src, dst, ex, sq, zeros_o)

    b1p = jnp.pad(b1, (0, CP - C_OUT)).reshape(1, CP)
    zout, loss = _tc4(oacc.reshape(NC, NP, CP), b1p, lp)
    return zout[:, :C_OUT], w, loss[0, 0]


# speedup vs baseline: 7.0667x; 1.0128x over previous
import functools

import jax
import jax.numpy as jnp
from jax import lax
from jax.experimental import pallas as pl
from jax.experimental.pallas import tpu as pltpu
from jax.experimental.pallas import tpu_sc as plsc

N = 10000
E = 320000
D = 128
HG = 32
H = 128
C_OUT = 40
CP = 48
ZW = 144
LAMB1 = 0.01
LAMB2 = 0.0001

NC = 2
NS = 16
NW = NC * NS
EPW = E // NW
CHUNK = 80
NCH = EPW // CHUNK
NP = 10112
RPT = NP // NS
L = 16

_mesh = plsc.VectorSubcoreMesh(core_axis_name="c", subcore_axis_name="s")
_sc_params = pltpu.CompilerParams(use_tc_tiling_on_sc=False,
                                  needs_layout_passes=False)


def _tc1_body(x_ref, wgl_ref, w0_ref, h_ref, p_ref):
    xb = x_ref[...]
    h_ref[...] = jnp.dot(xb, wgl_ref[...], preferred_element_type=jnp.float32)
    p_ref[...] = jnp.dot(xb, w0_ref[...], preferred_element_type=jnp.float32)


def _tc1(x, W_gl, W0):
    BR = 1000
    return pl.pallas_call(
        _tc1_body,
        grid=(N // BR,),
        in_specs=[pl.BlockSpec((BR, D), lambda i: (i, 0)),
                  pl.BlockSpec((D, HG), lambda i: (0, 0)),
                  pl.BlockSpec((D, H), lambda i: (0, 0))],
        out_specs=[pl.BlockSpec((BR, HG), lambda i: (i, 0)),
                   pl.BlockSpec((BR, H), lambda i: (i, 0))],
        out_shape=[jax.ShapeDtypeStruct((N, HG), jnp.float32),
                   jax.ShapeDtypeStruct((N, H), jnp.float32)],
    )(x, W_gl, W0)


@functools.partial(
    pl.kernel,
    out_type=[
        jax.ShapeDtypeStruct((NC * NP, ZW), jnp.float32),
        jax.ShapeDtypeStruct((E + CHUNK,), jnp.float32),
        jax.ShapeDtypeStruct((E + CHUNK,), jnp.float32),
    ],
    mesh=_mesh,
    compiler_params=_sc_params,
    scratch_types=[
        pltpu.VMEM((HG,), jnp.float32),
        [pltpu.VMEM((CHUNK,), jnp.int32)] * 2,
        [pltpu.VMEM((CHUNK,), jnp.int32)] * 2,
        [pltpu.VMEM((CHUNK, HG), jnp.float32)] * 2,
        [pltpu.VMEM((CHUNK, HG), jnp.float32)] * 2,
        pltpu.VMEM((CHUNK, D), jnp.float32),
        pltpu.VMEM((CHUNK, ZW), jnp.float32),
        pltpu.VMEM((CHUNK,), jnp.float32),
        pltpu.VMEM((CHUNK,), jnp.float32),
        pltpu.VMEM_SHARED((NP, ZW), jnp.float32),
        [pltpu.SemaphoreType.DMA] * 2,
        pltpu.SemaphoreType.DMA,
        pltpu.SemaphoreType.DMA,
    ],
)
def _sc_pass1(h_hbm, p_hbm, src_hbm, dst_hbm, a_hbm, zero_hbm,
              zacc_hbm, ex_hbm, sq_hbm,
              a_v, idxs_v, idxd_v, hs_v, hd_v, ps_v, pw_v, exb_v, sqb_v,
              zsh, sem, sem_ps, sem_sc):
    c = lax.axis_index("c")
    s = lax.axis_index("s")
    wid = s * NC + c
    r0 = s * RPT
    pltpu.sync_copy(zero_hbm.at[pl.ds(r0, RPT)], zsh.at[pl.ds(r0, RPT)])
    pltpu.sync_copy(a_hbm, a_v)
    plsc.subcore_barrier()

    iota = lax.iota(jnp.int32, L)
    av = [a_v[pl.ds(kk * L, L)] for kk in range(HG // L)]
    base0 = wid * EPW

    def prefetch(ch, b):
        base = base0 + ch * CHUNK
        pltpu.sync_copy(src_hbm.at[pl.ds(base, CHUNK)], idxs_v[b])
        pltpu.sync_copy(dst_hbm.at[pl.ds(base, CHUNK)], idxd_v[b])
        pltpu.async_copy(h_hbm.at[idxs_v[b]], hs_v[b], sem[b])
        pltpu.async_copy(h_hbm.at[idxd_v[b]], hd_v[b], sem[b])

    def wait_gathers(b):
        pltpu.make_async_copy(h_hbm.at[idxs_v[b]], hs_v[b], sem[b]).wait()
        pltpu.make_async_copy(h_hbm.at[idxd_v[b]], hd_v[b], sem[b]).wait()

    prefetch(0, 0)

    def do_chunk(ch, b, first):
        base = base0 + ch * CHUNK
        wait_gathers(b)
        cp_ps = pltpu.async_copy(p_hbm.at[idxs_v[b]], ps_v, sem_ps)
        if not first:
            pltpu.make_async_copy(pw_v, zsh.at[idxd_v[1 - b]], sem_sc).wait()
        prefetch(ch + 1, 1 - b)

        def group_body(g, carry):
            rows = g * L + iota
            e_acc = jnp.zeros((L,), jnp.float32)
            q_acc = jnp.zeros((L,), jnp.float32)
            for k in range(HG):
                colk = jnp.full((L,), k, jnp.int32)
                dk = (plsc.load_gather(hs_v[b], [rows, colk])
                      - plsc.load_gather(hd_v[b], [rows, colk]))
                e_acc = e_acc + av[k // L][k % L] * jnp.abs(dk)
                q_acc = q_acc + dk * dk
            ex = jnp.exp(jnp.maximum(e_acc, 0.0))
            exb_v[pl.ds(g * L, L)] = ex
            sqb_v[pl.ds(g * L, L)] = q_acc
            return carry

        lax.fori_loop(0, CHUNK // L, group_body, 0)
        cp_ps.wait()

        def weight_body(g, carry):
            ex = exb_v[pl.ds(g * L, L)]
            for r in range(L):
                row = g * L + r
                exi = ex[r]
                for j in range(D // L):
                    pw_v[row, pl.ds(j * L, L)] = ps_v[row, pl.ds(j * L, L)] * exi
                pw_v[row, pl.ds(D, L)] = jnp.where(iota == 0, exi, 0.0)
            return carry

        lax.fori_loop(0, CHUNK // L, weight_body, 0)

        pltpu.async_copy(pw_v, zsh.at[idxd_v[b]], sem_sc, add=True)
        pltpu.sync_copy(exb_v, ex_hbm.at[pl.ds(base, CHUNK)])
        pltpu.sync_copy(sqb_v, sq_hbm.at[pl.ds(base, CHUNK)])

    do_chunk(0, 0, True)
    do_chunk(1, 1, False)

    def pair_body2(i, carry):
        do_chunk(2 * i, 0, False)
        do_chunk(2 * i + 1, 1, False)
        return carry

    lax.fori_loop(1, NCH // 2, pair_body2, 0)
    do_chunk(NCH - 1, 0, False)
    pltpu.make_async_copy(pw_v, zsh.at[idxd_v[0]], sem_sc).wait()
    wait_gathers(1)
    plsc.subcore_barrier()
    pltpu.sync_copy(zsh.at[pl.ds(r0, RPT)],
                    zacc_hbm.at[pl.ds(c * NP + r0, RPT)])


def _tc2_body(zacc_ref, b0_ref, w1_ref, q_ref, sinv_ref):
    za = zacc_ref[0] + zacc_ref[1]
    sden = za[:, D:D + 1]
    sinv = 1.0 / (sden + 1e-16)
    z = jnp.maximum(za[:, :D] * sinv + b0_ref[...], 0.0)
    q_ref[...] = jnp.dot(z, w1_ref[...], preferred_element_type=jnp.float32)
    sinv_ref[...] = sinv


def _tc2(zacc3, b0r, W1p):
    BR = 1000
    return pl.pallas_call(
        _tc2_body,
        grid=(N // BR,),
        in_specs=[pl.BlockSpec((2, BR, ZW), lambda i: (0, i, 0)),
                  pl.BlockSpec((1, D), lambda i: (0, 0)),
                  pl.BlockSpec((D, CP), lambda i: (0, 0))],
        out_specs=[pl.BlockSpec((BR, CP), lambda i: (i, 0)),
                   pl.BlockSpec((BR, 1), lambda i: (i, 0))],
        out_shape=[jax.ShapeDtypeStruct((N, CP), jnp.float32),
                   jax.ShapeDtypeStruct((N, 1), jnp.float32)],
    )(zacc3, b0r, W1p)


@functools.partial(
    pl.kernel,
    out_type=[
        jax.ShapeDtypeStruct((E,), jnp.float32),
        jax.ShapeDtypeStruct((NC * NP, CP), jnp.float32),
        jax.ShapeDtypeStruct((2 * NW, L), jnp.float32),
    ],
    mesh=_mesh,
    compiler_params=_sc_params,
    scratch_types=[
        pltpu.VMEM((N,), jnp.float32),
        [pltpu.VMEM((CHUNK,), jnp.int32)] * 2,
        [pltpu.VMEM((CHUNK,), jnp.int32)] * 2,
        [pltpu.VMEM((CHUNK,), jnp.float32)] * 2,
        [pltpu.VMEM((CHUNK,), jnp.float32)] * 2,
        pltpu.VMEM((CHUNK,), jnp.float32),
        [pltpu.VMEM((CHUNK, CP), jnp.float32)] * 2,
        pltpu.VMEM((CHUNK, CP), jnp.float32),
        pltpu.VMEM((2, L), jnp.float32),
        pltpu.VMEM_SHARED((NP, CP), jnp.float32),
        [pltpu.SemaphoreType.DMA] * 2,
        pltpu.SemaphoreType.DMA,
    ],
)
def _sc_pass3(q_hbm, sinv_hbm, src_hbm, dst_hbm, ex_hbm, sq_hbm, zero_hbm,
              w_hbm, oacc_hbm, lp_hbm,
              sinv_v, idxs_v, idxd_v, exb_v, sqb_v, wb_v, qs_v, qw_v, lbuf,
              osh, sem, sem_sc):
    c = lax.axis_index("c")
    s = lax.axis_index("s")
    wid = s * NC + c
    r0 = s * RPT
    pltpu.sync_copy(zero_hbm.at[pl.ds(r0, RPT)], osh.at[pl.ds(r0, RPT)])
    pltpu.sync_copy(sinv_hbm, sinv_v)
    plsc.subcore_barrier()
    base0 = wid * EPW

    def prefetch(ch, b):
        base = base0 + ch * CHUNK
        pltpu.sync_copy(src_hbm.at[pl.ds(base, CHUNK)], idxs_v[b])
        pltpu.sync_copy(dst_hbm.at[pl.ds(base, CHUNK)], idxd_v[b])
        pltpu.sync_copy(ex_hbm.at[pl.ds(base, CHUNK)], exb_v[b])
        pltpu.sync_copy(sq_hbm.at[pl.ds(base, CHUNK)], sqb_v[b])
        pltpu.async_copy(q_hbm.at[idxs_v[b]], qs_v[b], sem[b])

    def wait_gathers(b):
        pltpu.make_async_copy(q_hbm.at[idxs_v[b]], qs_v[b], sem[b]).wait()

    prefetch(0, 0)

    def do_chunk(ch, b, carry, first=False):
        l1, l2 = carry
        base = base0 + ch * CHUNK
        wait_gathers(b)
        if not first:
            pltpu.make_async_copy(qw_v, osh.at[idxd_v[1 - b]], sem_sc).wait()
        prefetch(ch + 1, 1 - b)

        def group_body(g, car):
            a1, a2 = car
            sl = pl.ds(g * L, L)
            dvec = idxd_v[b][sl]
            sv = plsc.load_gather(sinv_v, [dvec])
            wv = exb_v[b][sl] * sv
            wb_v[sl] = wv
            for r in range(L):
                row = g * L + r
                wi = wv[r]
                for j in range(CP // L):
                    qw_v[row, pl.ds(j * L, L)] = qs_v[b][row, pl.ds(j * L, L)] * wi
            return (a1 + wv * sqb_v[b][sl], a2 + wv * wv)

        l1, l2 = lax.fori_loop(0, CHUNK // L, group_body, (l1, l2))

        pltpu.async_copy(qw_v, osh.at[idxd_v[b]], sem_sc, add=True)
        pltpu.sync_copy(wb_v, w_hbm.at[pl.ds(base, CHUNK)])
        return (l1, l2)

    def pair_body(i, carry):
        carry = do_chunk(2 * i, 0, carry)
        carry = do_chunk(2 * i + 1, 1, carry)
        return carry

    zl = jnp.zeros((L,), jnp.float32)
    carry = do_chunk(0, 0, (zl, zl), first=True)
    carry = do_chunk(1, 1, carry)
    l1, l2 = lax.fori_loop(1, NCH // 2, pair_body, carry)
    l1, l2 = do_chunk(NCH - 1, 0, (l1, l2))
    pltpu.make_async_copy(qw_v, osh.at[idxd_v[0]], sem_sc).wait()
    wait_gathers(1)
    lbuf[0, :] = l1
    lbuf[1, :] = l2
    plsc.subcore_barrier()
    pltpu.sync_copy(osh.at[pl.ds(r0, RPT)],
                    oacc_hbm.at[pl.ds(c * NP + r0, RPT)])
    pltpu.sync_copy(lbuf, lp_hbm.at[pl.ds(wid * 2, 2)])


def _tc4_body(oacc_ref, b1_ref, lp_ref, z_ref, loss_ref):
    z_ref[...] = oacc_ref[0] + oacc_ref[1] + b1_ref[...]

    @pl.when(pl.program_id(0) == 0)
    def _():
        lp = lp_ref[...]
        rid = lax.broadcasted_iota(jnp.int32, lp.shape, 0)
        l1 = jnp.sum(jnp.where(rid % 2 == 0, lp, 0.0))
        l2 = jnp.sum(jnp.where(rid % 2 == 1, lp, 0.0))
        loss_ref[...] = jnp.reshape(LAMB1 * 0.5 * l1 + LAMB2 * l2, (1, 1))


def _tc4(oacc3, b1r, lp):
    BR = 1000
    return pl.pallas_call(
        _tc4_body,
        grid=(N // BR,),
        in_specs=[pl.BlockSpec((2, BR, CP), lambda i: (0, i, 0)),
                  pl.BlockSpec((1, CP), lambda i: (0, 0)),
                  pl.BlockSpec((2 * NW, L), lambda i: (0, 0))],
        out_specs=[pl.BlockSpec((BR, CP), lambda i: (i, 0)),
                   pl.BlockSpec((1, 1), lambda i: (0, 0))],
        out_shape=[jax.ShapeDtypeStruct((N, CP), jnp.float32),
                   jax.ShapeDtypeStruct((1, 1), jnp.float32)],
    )(oacc3, b1r, lp)


def kernel(x, edge_index, W_gl, a_gl, W0, b0, W1, b1):
    pad = jnp.zeros((2, CHUNK), jnp.int32)
    eip = jnp.concatenate([edge_index, pad], axis=1)
    src = eip[0]
    dst = eip[1]
    h, p = _tc1(x, W_gl, W0)

    zeros_z = jnp.zeros((NP, ZW), jnp.float32)
    zacc, ex, sq = _sc_pass1(h, p, src, dst, a_gl, zeros_z)

    W1p = jnp.pad(W1, ((0, 0), (0, CP - C_OUT)))
    q, sinv = _tc2(zacc.reshape(NC, NP, ZW), b0.reshape(1, D), W1p)

    zeros_o = jnp.zeros((NP, CP), jnp.float32)
    w, oacc, lp = _sc_pass3(q, sinv.reshape(N), src, dst, ex, sq, zeros_o)

    b1p = jnp.pad(b1, (0, CP - C_OUT)).reshape(1, CP)
    zout, loss = _tc4(oacc.reshape(NC, NP, CP), b1p, lp)
    return zout[:, :C_OUT], w, loss[0, 0]


# speedup vs baseline: 8.7284x; 1.2351x over previous
import functools

import jax
import jax.numpy as jnp
from jax import lax
from jax.experimental import pallas as pl
from jax.experimental.pallas import tpu as pltpu
from jax.experimental.pallas import tpu_sc as plsc

N = 10000
E = 320000
D = 128
HG = 32
H = 128
C_OUT = 40
CP = 48
ZW = 144
LAMB1 = 0.01
LAMB2 = 0.0001

NC = 2
NS = 16
NW = NC * NS
EPW = E // NW
CHUNK = 80
NCH = EPW // CHUNK
NP = 10112
RPT = NP // NS
L = 16

_mesh = plsc.VectorSubcoreMesh(core_axis_name="c", subcore_axis_name="s")
_sc_params = pltpu.CompilerParams(use_tc_tiling_on_sc=False,
                                  needs_layout_passes=False)


def _tc1_body(x_ref, wgl_ref, w0_ref, h_ref, p_ref):
    xb = x_ref[...]
    h_ref[...] = jnp.dot(xb, wgl_ref[...], preferred_element_type=jnp.float32)
    p_ref[...] = jnp.dot(xb, w0_ref[...], preferred_element_type=jnp.float32)


def _tc1(x, W_gl, W0):
    BR = 1000
    return pl.pallas_call(
        _tc1_body,
        grid=(N // BR,),
        in_specs=[pl.BlockSpec((BR, D), lambda i: (i, 0)),
                  pl.BlockSpec((D, HG), lambda i: (0, 0)),
                  pl.BlockSpec((D, H), lambda i: (0, 0))],
        out_specs=[pl.BlockSpec((BR, HG), lambda i: (i, 0)),
                   pl.BlockSpec((BR, H), lambda i: (i, 0))],
        out_shape=[jax.ShapeDtypeStruct((N, HG), jnp.float32),
                   jax.ShapeDtypeStruct((N, H), jnp.float32)],
    )(x, W_gl, W0)


@functools.partial(
    pl.kernel,
    out_type=[
        jax.ShapeDtypeStruct((NC * NP, ZW), jnp.float32),
        jax.ShapeDtypeStruct((E + CHUNK,), jnp.float32),
        jax.ShapeDtypeStruct((E + CHUNK,), jnp.float32),
    ],
    mesh=_mesh,
    compiler_params=_sc_params,
    scratch_types=[
        pltpu.VMEM((HG,), jnp.float32),
        [pltpu.VMEM((CHUNK,), jnp.int32)] * 2,
        [pltpu.VMEM((CHUNK,), jnp.int32)] * 2,
        [pltpu.VMEM((CHUNK, HG), jnp.float32)] * 2,
        [pltpu.VMEM((CHUNK, HG), jnp.float32)] * 2,
        pltpu.VMEM((CHUNK, D), jnp.float32),
        pltpu.VMEM((CHUNK, ZW), jnp.float32),
        pltpu.VMEM((CHUNK,), jnp.float32),
        pltpu.VMEM((CHUNK,), jnp.float32),
        pltpu.VMEM_SHARED((NP, ZW), jnp.float32),
        [pltpu.SemaphoreType.DMA] * 2,
        pltpu.SemaphoreType.DMA,
        pltpu.SemaphoreType.DMA,
    ],
)
def _sc_pass1(h_hbm, p_hbm, src_hbm, dst_hbm, a_hbm, zero_hbm,
              zacc_hbm, ex_hbm, sq_hbm,
              a_v, idxs_v, idxd_v, hs_v, hd_v, ps_v, pw_v, exb_v, sqb_v,
              zsh, sem, sem_ps, sem_sc):
    c = lax.axis_index("c")
    s = lax.axis_index("s")
    wid = s * NC + c
    r0 = s * RPT
    pltpu.sync_copy(zero_hbm.at[pl.ds(r0, RPT)], zsh.at[pl.ds(r0, RPT)])
    pltpu.sync_copy(a_hbm, a_v)
    plsc.subcore_barrier()

    iota = lax.iota(jnp.int32, L)
    av = [a_v[pl.ds(kk * L, L)] for kk in range(HG // L)]
    base0 = wid * EPW

    def prefetch(ch, b):
        base = base0 + ch * CHUNK
        pltpu.sync_copy(src_hbm.at[pl.ds(base, CHUNK)], idxs_v[b])
        pltpu.sync_copy(dst_hbm.at[pl.ds(base, CHUNK)], idxd_v[b])
        pltpu.async_copy(h_hbm.at[idxs_v[b]], hs_v[b], sem[b])
        pltpu.async_copy(h_hbm.at[idxd_v[b]], hd_v[b], sem[b])

    def wait_gathers(b):
        pltpu.make_async_copy(h_hbm.at[idxs_v[b]], hs_v[b], sem[b]).wait()
        pltpu.make_async_copy(h_hbm.at[idxd_v[b]], hd_v[b], sem[b]).wait()

    prefetch(0, 0)

    def do_chunk(ch, b, first):
        base = base0 + ch * CHUNK
        wait_gathers(b)
        cp_ps = pltpu.async_copy(p_hbm.at[idxs_v[b]], ps_v, sem_ps)
        if not first:
            pltpu.make_async_copy(pw_v, zsh.at[idxd_v[1 - b]], sem_sc).wait()
        prefetch(ch + 1, 1 - b)

        def group_body(g, carry):
            e_acc = jnp.zeros((L,), jnp.float32)
            q_acc = jnp.zeros((L,), jnp.float32)
            for r in range(L):
                row = g * L + r
                d0 = hs_v[b][row, pl.ds(0, L)] - hd_v[b][row, pl.ds(0, L)]
                d1 = hs_v[b][row, pl.ds(L, L)] - hd_v[b][row, pl.ds(L, L)]
                t = av[0] * jnp.abs(d0) + av[1] * jnp.abs(d1)
                q = d0 * d0 + d1 * d1
                e_r = jnp.sum(t)
                q_r = jnp.sum(q)
                e_acc = jnp.where(iota == r, e_r, e_acc)
                q_acc = jnp.where(iota == r, q_r, q_acc)
            ex = jnp.exp(jnp.maximum(e_acc, 0.0))
            exb_v[pl.ds(g * L, L)] = ex
            sqb_v[pl.ds(g * L, L)] = q_acc
            return carry

        lax.fori_loop(0, CHUNK // L, group_body, 0)
        cp_ps.wait()

        def weight_body(g, carry):
            ex = exb_v[pl.ds(g * L, L)]
            for r in range(L):
                row = g * L + r
                exi = ex[r]
                for j in range(D // L):
                    pw_v[row, pl.ds(j * L, L)] = ps_v[row, pl.ds(j * L, L)] * exi
                pw_v[row, pl.ds(D, L)] = jnp.where(iota == 0, exi, 0.0)
            return carry

        lax.fori_loop(0, CHUNK // L, weight_body, 0)

        pltpu.async_copy(pw_v, zsh.at[idxd_v[b]], sem_sc, add=True)
        pltpu.sync_copy(exb_v, ex_hbm.at[pl.ds(base, CHUNK)])
        pltpu.sync_copy(sqb_v, sq_hbm.at[pl.ds(base, CHUNK)])

    do_chunk(0, 0, True)
    do_chunk(1, 1, False)

    def pair_body2(i, carry):
        do_chunk(2 * i, 0, False)
        do_chunk(2 * i + 1, 1, False)
        return carry

    lax.fori_loop(1, NCH // 2, pair_body2, 0)
    do_chunk(NCH - 1, 0, False)
    pltpu.make_async_copy(pw_v, zsh.at[idxd_v[0]], sem_sc).wait()
    wait_gathers(1)
    plsc.subcore_barrier()
    pltpu.sync_copy(zsh.at[pl.ds(r0, RPT)],
                    zacc_hbm.at[pl.ds(c * NP + r0, RPT)])


def _tc2_body(zacc_ref, b0_ref, w1_ref, q_ref, sinv_ref):
    za = zacc_ref[0] + zacc_ref[1]
    sden = za[:, D:D + 1]
    sinv = 1.0 / (sden + 1e-16)
    z = jnp.maximum(za[:, :D] * sinv + b0_ref[...], 0.0)
    q_ref[...] = jnp.dot(z, w1_ref[...], preferred_element_type=jnp.float32)
    sinv_ref[...] = sinv


def _tc2(zacc3, b0r, W1p):
    BR = 1000
    return pl.pallas_call(
        _tc2_body,
        grid=(N // BR,),
        in_specs=[pl.BlockSpec((2, BR, ZW), lambda i: (0, i, 0)),
                  pl.BlockSpec((1, D), lambda i: (0, 0)),
                  pl.BlockSpec((D, CP), lambda i: (0, 0))],
        out_specs=[pl.BlockSpec((BR, CP), lambda i: (i, 0)),
                   pl.BlockSpec((BR, 1), lambda i: (i, 0))],
        out_shape=[jax.ShapeDtypeStruct((N, CP), jnp.float32),
                   jax.ShapeDtypeStruct((N, 1), jnp.float32)],
    )(zacc3, b0r, W1p)


@functools.partial(
    pl.kernel,
    out_type=[
        jax.ShapeDtypeStruct((E,), jnp.float32),
        jax.ShapeDtypeStruct((NC * NP, CP), jnp.float32),
        jax.ShapeDtypeStruct((2 * NW, L), jnp.float32),
    ],
    mesh=_mesh,
    compiler_params=_sc_params,
    scratch_types=[
        pltpu.VMEM((N,), jnp.float32),
        [pltpu.VMEM((CHUNK,), jnp.int32)] * 2,
        [pltpu.VMEM((CHUNK,), jnp.int32)] * 2,
        [pltpu.VMEM((CHUNK,), jnp.float32)] * 2,
        [pltpu.VMEM((CHUNK,), jnp.float32)] * 2,
        pltpu.VMEM((CHUNK,), jnp.float32),
        [pltpu.VMEM((CHUNK, CP), jnp.float32)] * 2,
        pltpu.VMEM((CHUNK, CP), jnp.float32),
        pltpu.VMEM((2, L), jnp.float32),
        pltpu.VMEM_SHARED((NP, CP), jnp.float32),
        [pltpu.SemaphoreType.DMA] * 2,
        pltpu.SemaphoreType.DMA,
    ],
)
def _sc_pass3(q_hbm, sinv_hbm, src_hbm, dst_hbm, ex_hbm, sq_hbm, zero_hbm,
              w_hbm, oacc_hbm, lp_hbm,
              sinv_v, idxs_v, idxd_v, exb_v, sqb_v, wb_v, qs_v, qw_v, lbuf,
              osh, sem, sem_sc):
    c = lax.axis_index("c")
    s = lax.axis_index("s")
    wid = s * NC + c
    r0 = s * RPT
    pltpu.sync_copy(zero_hbm.at[pl.ds(r0, RPT)], osh.at[pl.ds(r0, RPT)])
    pltpu.sync_copy(sinv_hbm, sinv_v)
    plsc.subcore_barrier()
    base0 = wid * EPW

    def prefetch(ch, b):
        base = base0 + ch * CHUNK
        pltpu.sync_copy(src_hbm.at[pl.ds(base, CHUNK)], idxs_v[b])
        pltpu.sync_copy(dst_hbm.at[pl.ds(base, CHUNK)], idxd_v[b])
        pltpu.sync_copy(ex_hbm.at[pl.ds(base, CHUNK)], exb_v[b])
        pltpu.sync_copy(sq_hbm.at[pl.ds(base, CHUNK)], sqb_v[b])
        pltpu.async_copy(q_hbm.at[idxs_v[b]], qs_v[b], sem[b])

    def wait_gathers(b):
        pltpu.make_async_copy(q_hbm.at[idxs_v[b]], qs_v[b], sem[b]).wait()

    prefetch(0, 0)

    def do_chunk(ch, b, carry, first=False):
        l1, l2 = carry
        base = base0 + ch * CHUNK
        wait_gathers(b)
        if not first:
            pltpu.make_async_copy(qw_v, osh.at[idxd_v[1 - b]], sem_sc).wait()
        prefetch(ch + 1, 1 - b)

        def group_body(g, car):
            a1, a2 = car
            sl = pl.ds(g * L, L)
            dvec = idxd_v[b][sl]
            sv = plsc.load_gather(sinv_v, [dvec])
            wv = exb_v[b][sl] * sv
            wb_v[sl] = wv
            for r in range(L):
                row = g * L + r
                wi = wv[r]
                for j in range(CP // L):
                    qw_v[row, pl.ds(j * L, L)] = qs_v[b][row, pl.ds(j * L, L)] * wi
            return (a1 + wv * sqb_v[b][sl], a2 + wv * wv)

        l1, l2 = lax.fori_loop(0, CHUNK // L, group_body, (l1, l2))

        pltpu.async_copy(qw_v, osh.at[idxd_v[b]], sem_sc, add=True)
        pltpu.sync_copy(wb_v, w_hbm.at[pl.ds(base, CHUNK)])
        return (l1, l2)

    def pair_body(i, carry):
        carry = do_chunk(2 * i, 0, carry)
        carry = do_chunk(2 * i + 1, 1, carry)
        return carry

    zl = jnp.zeros((L,), jnp.float32)
    carry = do_chunk(0, 0, (zl, zl), first=True)
    carry = do_chunk(1, 1, carry)
    l1, l2 = lax.fori_loop(1, NCH // 2, pair_body, carry)
    l1, l2 = do_chunk(NCH - 1, 0, (l1, l2))
    pltpu.make_async_copy(qw_v, osh.at[idxd_v[0]], sem_sc).wait()
    wait_gathers(1)
    lbuf[0, :] = l1
    lbuf[1, :] = l2
    plsc.subcore_barrier()
    pltpu.sync_copy(osh.at[pl.ds(r0, RPT)],
                    oacc_hbm.at[pl.ds(c * NP + r0, RPT)])
    pltpu.sync_copy(lbuf, lp_hbm.at[pl.ds(wid * 2, 2)])


def _tc4_body(oacc_ref, b1_ref, lp_ref, z_ref, loss_ref):
    z_ref[...] = oacc_ref[0] + oacc_ref[1] + b1_ref[...]

    @pl.when(pl.program_id(0) == 0)
    def _():
        lp = lp_ref[...]
        rid = lax.broadcasted_iota(jnp.int32, lp.shape, 0)
        l1 = jnp.sum(jnp.where(rid % 2 == 0, lp, 0.0))
        l2 = jnp.sum(jnp.where(rid % 2 == 1, lp, 0.0))
        loss_ref[...] = jnp.reshape(LAMB1 * 0.5 * l1 + LAMB2 * l2, (1, 1))


def _tc4(oacc3, b1r, lp):
    BR = 1000
    return pl.pallas_call(
        _tc4_body,
        grid=(N // BR,),
        in_specs=[pl.BlockSpec((2, BR, CP), lambda i: (0, i, 0)),
                  pl.BlockSpec((1, CP), lambda i: (0, 0)),
                  pl.BlockSpec((2 * NW, L), lambda i: (0, 0))],
        out_specs=[pl.BlockSpec((BR, CP), lambda i: (i, 0)),
                   pl.BlockSpec((1, 1), lambda i: (0, 0))],
        out_shape=[jax.ShapeDtypeStruct((N, CP), jnp.float32),
                   jax.ShapeDtypeStruct((1, 1), jnp.float32)],
    )(oacc3, b1r, lp)


def kernel(x, edge_index, W_gl, a_gl, W0, b0, W1, b1):
    pad = jnp.zeros((2, CHUNK), jnp.int32)
    eip = jnp.concatenate([edge_index, pad], axis=1)
    src = eip[0]
    dst = eip[1]
    h, p = _tc1(x, W_gl, W0)

    zeros_z = jnp.zeros((NP, ZW), jnp.float32)
    zacc, ex, sq = _sc_pass1(h, p, src, dst, a_gl, zeros_z)

    W1p = jnp.pad(W1, ((0, 0), (0, CP - C_OUT)))
    q, sinv = _tc2(zacc.reshape(NC, NP, ZW), b0.reshape(1, D), W1p)

    zeros_o = jnp.zeros((NP, CP), jnp.float32)
    w, oacc, lp = _sc_pass3(q, sinv.reshape(N), src, dst, ex, sq, zeros_o)

    b1p = jnp.pad(b1, (0, CP - C_OUT)).reshape(1, CP)
    zout, loss = _tc4(oacc.reshape(NC, NP, CP), b1p, lp)
    return zout[:, :C_OUT], w, loss[0, 0]


# speedup vs baseline: 10.1231x; 1.1598x over previous
import functools

import jax
import jax.numpy as jnp
from jax import lax
from jax.experimental import pallas as pl
from jax.experimental.pallas import tpu as pltpu
from jax.experimental.pallas import tpu_sc as plsc

N = 10000
E = 320000
D = 128
HG = 32
H = 128
C_OUT = 40
CP = 48
ZW = 144
LAMB1 = 0.01
LAMB2 = 0.0001

NC = 2
NS = 16
NW = NC * NS
EPW = E // NW
CHUNK = 80
NCH = EPW // CHUNK
NP = 10112
RPT = NP // NS
L = 16

_mesh = plsc.VectorSubcoreMesh(core_axis_name="c", subcore_axis_name="s")
_sc_params = pltpu.CompilerParams(use_tc_tiling_on_sc=False,
                                  needs_layout_passes=False)


def _tc1_body(x_ref, wgl_ref, w0_ref, h_ref, p_ref):
    xb = x_ref[...]
    h_ref[...] = jnp.dot(xb, wgl_ref[...], preferred_element_type=jnp.float32)
    p_ref[...] = jnp.dot(xb, w0_ref[...], preferred_element_type=jnp.float32)


def _tc1(x, W_gl, W0):
    BR = 1000
    return pl.pallas_call(
        _tc1_body,
        grid=(N // BR,),
        in_specs=[pl.BlockSpec((BR, D), lambda i: (i, 0)),
                  pl.BlockSpec((D, HG), lambda i: (0, 0)),
                  pl.BlockSpec((D, H), lambda i: (0, 0))],
        out_specs=[pl.BlockSpec((BR, HG), lambda i: (i, 0)),
                   pl.BlockSpec((BR, H), lambda i: (i, 0))],
        out_shape=[jax.ShapeDtypeStruct((N, HG), jnp.float32),
                   jax.ShapeDtypeStruct((N, H), jnp.float32)],
    )(x, W_gl, W0)


@functools.partial(
    pl.kernel,
    out_type=[
        jax.ShapeDtypeStruct((NC * NP, ZW), jnp.float32),
        jax.ShapeDtypeStruct((E + CHUNK,), jnp.float32),
        jax.ShapeDtypeStruct((E + CHUNK,), jnp.float32),
    ],
    mesh=_mesh,
    compiler_params=_sc_params,
    scratch_types=[
        pltpu.VMEM((HG,), jnp.float32),
        [pltpu.VMEM((2, CHUNK), jnp.int32)] * 2,
        [pltpu.VMEM((CHUNK, HG), jnp.float32)] * 2,
        [pltpu.VMEM((CHUNK, HG), jnp.float32)] * 2,
        pltpu.VMEM((CHUNK, D), jnp.float32),
        pltpu.VMEM((CHUNK, ZW), jnp.float32),
        pltpu.VMEM((2 * CHUNK,), jnp.float32),
        pltpu.VMEM((2 * CHUNK,), jnp.float32),
        pltpu.VMEM_SHARED((NP, ZW), jnp.float32),
        [pltpu.SemaphoreType.DMA] * 2,
        pltpu.SemaphoreType.DMA,
        pltpu.SemaphoreType.DMA,
    ],
)
def _sc_pass1(h_hbm, p_hbm, e2w_hbm, a_hbm, zero_hbm,
              zacc_hbm, ex_hbm, sq_hbm,
              a_v, ebuf, hs_v, hd_v, ps_v, pw_v, exb_v, sqb_v,
              zsh, sem, sem_ps, sem_sc):
    c = lax.axis_index("c")
    s = lax.axis_index("s")
    wid = s * NC + c
    r0 = s * RPT
    pltpu.sync_copy(zero_hbm.at[pl.ds(r0, RPT)], zsh.at[pl.ds(r0, RPT)])
    pltpu.sync_copy(a_hbm, a_v)
    plsc.subcore_barrier()

    iota = lax.iota(jnp.int32, L)
    av = [a_v[pl.ds(kk * L, L)] for kk in range(HG // L)]
    base0 = wid * EPW

    def prefetch(ch, b):
        pltpu.sync_copy(e2w_hbm.at[wid, ch], ebuf[b])
        pltpu.async_copy(h_hbm.at[ebuf[b].at[0]], hs_v[b], sem[b])
        pltpu.async_copy(h_hbm.at[ebuf[b].at[1]], hd_v[b], sem[b])

    def wait_gathers(b):
        pltpu.make_async_copy(h_hbm.at[ebuf[b].at[0]], hs_v[b], sem[b]).wait()
        pltpu.make_async_copy(h_hbm.at[ebuf[b].at[1]], hd_v[b], sem[b]).wait()

    prefetch(0, 0)

    def do_chunk(ch, b, first):
        wait_gathers(b)
        cp_ps = pltpu.async_copy(p_hbm.at[ebuf[b].at[0]], ps_v, sem_ps)
        if not first:
            pltpu.make_async_copy(pw_v, zsh.at[ebuf[1 - b].at[1]], sem_sc).wait()
        prefetch(ch + 1, 1 - b)

        def group_body(g, carry):
            e_acc = jnp.zeros((L,), jnp.float32)
            q_acc = jnp.zeros((L,), jnp.float32)
            for r in range(L):
                row = g * L + r
                d0 = hs_v[b][row, pl.ds(0, L)] - hd_v[b][row, pl.ds(0, L)]
                d1 = hs_v[b][row, pl.ds(L, L)] - hd_v[b][row, pl.ds(L, L)]
                t = av[0] * jnp.abs(d0) + av[1] * jnp.abs(d1)
                q = d0 * d0 + d1 * d1
                e_r = jnp.sum(t)
                q_r = jnp.sum(q)
                e_acc = jnp.where(iota == r, e_r, e_acc)
                q_acc = jnp.where(iota == r, q_r, q_acc)
            ex = jnp.exp(jnp.maximum(e_acc, 0.0))
            exb_v[pl.ds(b * CHUNK + g * L, L)] = ex
            sqb_v[pl.ds(b * CHUNK + g * L, L)] = q_acc
            return carry

        lax.fori_loop(0, CHUNK // L, group_body, 0)
        cp_ps.wait()

        def weight_body(g, carry):
            ex = exb_v[pl.ds(b * CHUNK + g * L, L)]
            for r in range(L):
                row = g * L + r
                exi = ex[r]
                for j in range(D // L):
                    pw_v[row, pl.ds(j * L, L)] = ps_v[row, pl.ds(j * L, L)] * exi
                pw_v[row, pl.ds(D, L)] = jnp.where(iota == 0, exi, 0.0)
            return carry

        lax.fori_loop(0, CHUNK // L, weight_body, 0)

        pltpu.async_copy(pw_v, zsh.at[ebuf[b].at[1]], sem_sc, add=True)

    def flush_pair(pair):
        base = base0 + pair * 2 * CHUNK
        pltpu.sync_copy(exb_v, ex_hbm.at[pl.ds(base, 2 * CHUNK)])
        pltpu.sync_copy(sqb_v, sq_hbm.at[pl.ds(base, 2 * CHUNK)])

    do_chunk(0, 0, True)
    do_chunk(1, 1, False)
    flush_pair(0)

    def pair_body2(i, carry):
        do_chunk(2 * i, 0, False)
        do_chunk(2 * i + 1, 1, False)
        flush_pair(i)
        return carry

    lax.fori_loop(1, NCH // 2, pair_body2, 0)
    do_chunk(NCH - 1, 0, False)
    pltpu.sync_copy(exb_v.at[pl.ds(0, CHUNK)],
                    ex_hbm.at[pl.ds(base0 + (NCH - 1) * CHUNK, CHUNK)])
    pltpu.sync_copy(sqb_v.at[pl.ds(0, CHUNK)],
                    sq_hbm.at[pl.ds(base0 + (NCH - 1) * CHUNK, CHUNK)])
    pltpu.make_async_copy(pw_v, zsh.at[ebuf[0].at[1]], sem_sc).wait()
    wait_gathers(1)
    plsc.subcore_barrier()
    pltpu.sync_copy(zsh.at[pl.ds(r0, RPT)],
                    zacc_hbm.at[pl.ds(c * NP + r0, RPT)])


def _tc2_body(zacc_ref, b0_ref, w1_ref, q_ref, sinv_ref):
    za = zacc_ref[0] + zacc_ref[1]
    sden = za[:, D:D + 1]
    sinv = 1.0 / (sden + 1e-16)
    z = jnp.maximum(za[:, :D] * sinv + b0_ref[...], 0.0)
    q_ref[...] = jnp.dot(z, w1_ref[...], preferred_element_type=jnp.float32)
    sinv_ref[...] = sinv


def _tc2(zacc3, b0r, W1p):
    BR = 1000
    return pl.pallas_call(
        _tc2_body,
        grid=(N // BR,),
        in_specs=[pl.BlockSpec((2, BR, ZW), lambda i: (0, i, 0)),
                  pl.BlockSpec((1, D), lambda i: (0, 0)),
                  pl.BlockSpec((D, CP), lambda i: (0, 0))],
        out_specs=[pl.BlockSpec((BR, CP), lambda i: (i, 0)),
                   pl.BlockSpec((BR, 1), lambda i: (i, 0))],
        out_shape=[jax.ShapeDtypeStruct((N, CP), jnp.float32),
                   jax.ShapeDtypeStruct((N, 1), jnp.float32)],
    )(zacc3, b0r, W1p)


@functools.partial(
    pl.kernel,
    out_type=[
        jax.ShapeDtypeStruct((E,), jnp.float32),
        jax.ShapeDtypeStruct((NC * NP, CP), jnp.float32),
        jax.ShapeDtypeStruct((2 * NW, L), jnp.float32),
    ],
    mesh=_mesh,
    compiler_params=_sc_params,
    scratch_types=[
        pltpu.VMEM((N,), jnp.float32),
        pltpu.VMEM((NCH + 1, CHUNK), jnp.int32),
        pltpu.VMEM((NCH + 1, CHUNK), jnp.int32),
        pltpu.VMEM((EPW,), jnp.float32),
        pltpu.VMEM((EPW,), jnp.float32),
        pltpu.VMEM((EPW,), jnp.float32),
        [pltpu.VMEM((CHUNK, CP), jnp.float32)] * 2,
        pltpu.VMEM((CHUNK, CP), jnp.float32),
        pltpu.VMEM((2, L), jnp.float32),
        pltpu.VMEM_SHARED((NP, CP), jnp.float32),
        [pltpu.SemaphoreType.DMA] * 2,
        pltpu.SemaphoreType.DMA,
    ],
)
def _sc_pass3(q_hbm, sinv_hbm, srcw_hbm, dstw_hbm, exw_hbm, sqw_hbm, zero_hbm,
              w_hbm, oacc_hbm, lp_hbm,
              sinv_v, idxs_a, idxd_a, exa, sqa, wba, qs_v, qw_v, lbuf,
              osh, sem, sem_sc):
    c = lax.axis_index("c")
    s = lax.axis_index("s")
    wid = s * NC + c
    r0 = s * RPT
    pltpu.sync_copy(zero_hbm.at[pl.ds(r0, RPT)], osh.at[pl.ds(r0, RPT)])
    pltpu.sync_copy(sinv_hbm, sinv_v)
    pltpu.sync_copy(srcw_hbm.at[wid], idxs_a)
    pltpu.sync_copy(dstw_hbm.at[wid], idxd_a)
    pltpu.sync_copy(exw_hbm.at[wid], exa)
    pltpu.sync_copy(sqw_hbm.at[wid], sqa)
    plsc.subcore_barrier()

    def prefetch(ch, b):
        pltpu.async_copy(q_hbm.at[idxs_a.at[ch]], qs_v[b], sem[b])

    def wait_gathers(ch, b):
        pltpu.make_async_copy(q_hbm.at[idxs_a.at[ch]], qs_v[b], sem[b]).wait()

    prefetch(0, 0)

    def do_chunk(ch, b, carry, first=False):
        l1, l2 = carry
        wait_gathers(ch, b)
        if not first:
            pltpu.make_async_copy(qw_v, osh.at[idxd_a.at[ch - 1]],
                                  sem_sc).wait()
        prefetch(ch + 1, 1 - b)
        eoff = ch * CHUNK

        def group_body(g, car):
            a1, a2 = car
            esl = pl.ds(eoff + g * L, L)
            dvec = idxd_a[ch, pl.ds(g * L, L)]
            sv = plsc.load_gather(sinv_v, [dvec])
            wv = exa[esl] * sv
            wba[esl] = wv
            for r in range(L):
                row = g * L + r
                wi = wv[r]
                for j in range(CP // L):
                    qw_v[row, pl.ds(j * L, L)] = qs_v[b][row, pl.ds(j * L, L)] * wi
            return (a1 + wv * sqa[esl], a2 + wv * wv)

        l1, l2 = lax.fori_loop(0, CHUNK // L, group_body, (l1, l2))
        pltpu.async_copy(qw_v, osh.at[idxd_a.at[ch]], sem_sc, add=True)
        return (l1, l2)

    def pair_body(i, carry):
        carry = do_chunk(2 * i, 0, carry)
        carry = do_chunk(2 * i + 1, 1, carry)
        return carry

    zl = jnp.zeros((L,), jnp.float32)
    carry = do_chunk(0, 0, (zl, zl), first=True)
    carry = do_chunk(1, 1, carry)
    l1, l2 = lax.fori_loop(1, NCH // 2, pair_body, carry)
    l1, l2 = do_chunk(NCH - 1, 0, (l1, l2))
    pltpu.make_async_copy(qw_v, osh.at[idxd_a.at[NCH - 1]], sem_sc).wait()
    wait_gathers(NCH, 1)
    pltpu.sync_copy(wba, w_hbm.at[pl.ds(wid * EPW, EPW)])
    lbuf[0, :] = l1
    lbuf[1, :] = l2
    plsc.subcore_barrier()
    pltpu.sync_copy(osh.at[pl.ds(r0, RPT)],
                    oacc_hbm.at[pl.ds(c * NP + r0, RPT)])
    pltpu.sync_copy(lbuf, lp_hbm.at[pl.ds(wid * 2, 2)])


def _tc4_body(oacc_ref, b1_ref, lp_ref, z_ref, loss_ref):
    z_ref[...] = oacc_ref[0] + oacc_ref[1] + b1_ref[...]

    @pl.when(pl.program_id(0) == 0)
    def _():
        lp = lp_ref[...]
        rid = lax.broadcasted_iota(jnp.int32, lp.shape, 0)
        l1 = jnp.sum(jnp.where(rid % 2 == 0, lp, 0.0))
        l2 = jnp.sum(jnp.where(rid % 2 == 1, lp, 0.0))
        loss_ref[...] = jnp.reshape(LAMB1 * 0.5 * l1 + LAMB2 * l2, (1, 1))


def _tc4(oacc3, b1r, lp):
    BR = 1000
    return pl.pallas_call(
        _tc4_body,
        grid=(N // BR,),
        in_specs=[pl.BlockSpec((2, BR, CP), lambda i: (0, i, 0)),
                  pl.BlockSpec((1, CP), lambda i: (0, 0)),
                  pl.BlockSpec((2 * NW, L), lambda i: (0, 0))],
        out_specs=[pl.BlockSpec((BR, CP), lambda i: (i, 0)),
                   pl.BlockSpec((1, 1), lambda i: (0, 0))],
        out_shape=[jax.ShapeDtypeStruct((N, CP), jnp.float32),
                   jax.ShapeDtypeStruct((1, 1), jnp.float32)],
    )(oacc3, b1r, lp)


def kernel(x, edge_index, W_gl, a_gl, W0, b0, W1, b1):
    h, p = _tc1(x, W_gl, W0)

    zeros_z = jnp.zeros((NP, ZW), jnp.float32)
    srcw = jnp.pad(edge_index[0].reshape(NW, NCH, CHUNK),
                   ((0, 0), (0, 1), (0, 0)))
    dstw = jnp.pad(edge_index[1].reshape(NW, NCH, CHUNK),
                   ((0, 0), (0, 1), (0, 0)))
    e2w = jnp.stack([srcw, dstw], axis=2)
    zacc, ex, sq = _sc_pass1(h, p, e2w, a_gl, zeros_z)

    W1p = jnp.pad(W1, ((0, 0), (0, CP - C_OUT)))
    q, sinv = _tc2(zacc.reshape(NC, NP, ZW), b0.reshape(1, D), W1p)

    zeros_o = jnp.zeros((NP, CP), jnp.float32)
    exw = ex[:E].reshape(NW, EPW)
    sqw = sq[:E].reshape(NW, EPW)
    w, oacc, lp = _sc_pass3(q, sinv.reshape(N), srcw, dstw, exw, sqw, zeros_o)

    b1p = jnp.pad(b1, (0, CP - C_OUT)).reshape(1, CP)
    zout, loss = _tc4(oacc.reshape(NC, NP, CP), b1p, lp)
    return zout[:, :C_OUT], w, loss[0, 0]


# speedup vs baseline: 12.9610x; 1.2803x over previous
import functools

import jax
import jax.numpy as jnp
from jax import lax
from jax.experimental import pallas as pl
from jax.experimental.pallas import tpu as pltpu
from jax.experimental.pallas import tpu_sc as plsc

N = 10000
E = 320000
D = 128
HG = 32
H = 128
C_OUT = 40
CP = 48
ZW = 144
LAMB1 = 0.01
LAMB2 = 0.0001

NC = 2
NS = 16
NW = NC * NS
EPW = E // NW
CHUNK = 80
NCH = EPW // CHUNK
NP = 10112
RPT = NP // NS
L = 16

_mesh = plsc.VectorSubcoreMesh(core_axis_name="c", subcore_axis_name="s")
_sc_params = pltpu.CompilerParams(use_tc_tiling_on_sc=False,
                                  needs_layout_passes=False)


def _tc1_body(x_ref, wgl_ref, w0_ref, h_ref, p_ref):
    xb = x_ref[...]
    h_ref[...] = jnp.dot(xb, wgl_ref[...], preferred_element_type=jnp.float32)
    p_ref[...] = jnp.dot(xb, w0_ref[...], preferred_element_type=jnp.float32)


def _tc1(x, W_gl, W0):
    BR = 1000
    return pl.pallas_call(
        _tc1_body,
        grid=(N // BR,),
        in_specs=[pl.BlockSpec((BR, D), lambda i: (i, 0)),
                  pl.BlockSpec((D, HG), lambda i: (0, 0)),
                  pl.BlockSpec((D, H), lambda i: (0, 0))],
        out_specs=[pl.BlockSpec((BR, HG), lambda i: (i, 0)),
                   pl.BlockSpec((BR, H), lambda i: (i, 0))],
        out_shape=[jax.ShapeDtypeStruct((N, HG), jnp.float32),
                   jax.ShapeDtypeStruct((N, H), jnp.float32)],
    )(x, W_gl, W0)


@functools.partial(
    pl.kernel,
    out_type=[
        jax.ShapeDtypeStruct((NC * NP, ZW), jnp.float32),
        jax.ShapeDtypeStruct((E + CHUNK,), jnp.float32),
        jax.ShapeDtypeStruct((E + CHUNK,), jnp.float32),
    ],
    mesh=_mesh,
    compiler_params=_sc_params,
    scratch_types=[
        pltpu.VMEM((HG,), jnp.float32),
        [pltpu.VMEM((2, CHUNK), jnp.int32)] * 2,
        [pltpu.VMEM((CHUNK, HG), jnp.float32)] * 2,
        [pltpu.VMEM((CHUNK, HG), jnp.float32)] * 2,
        pltpu.VMEM((CHUNK, D), jnp.float32),
        pltpu.VMEM((CHUNK, ZW), jnp.float32),
        pltpu.VMEM((2 * CHUNK,), jnp.float32),
        pltpu.VMEM((2 * CHUNK,), jnp.float32),
        pltpu.VMEM_SHARED((NP, ZW), jnp.float32),
        [pltpu.SemaphoreType.DMA] * 2,
        pltpu.SemaphoreType.DMA,
        pltpu.SemaphoreType.DMA,
    ],
)
def _sc_pass1(h_hbm, p_hbm, e2w_hbm, a_hbm, zero_hbm,
              zacc_hbm, ex_hbm, sq_hbm,
              a_v, ebuf, hs_v, hd_v, ps_v, pw_v, exb_v, sqb_v,
              zsh, sem, sem_ps, sem_sc):
    c = lax.axis_index("c")
    s = lax.axis_index("s")
    wid = s * NC + c
    r0 = s * RPT
    pltpu.sync_copy(zero_hbm.at[pl.ds(r0, RPT)], zsh.at[pl.ds(r0, RPT)])
    pltpu.sync_copy(a_hbm, a_v)
    plsc.subcore_barrier()

    iota = lax.iota(jnp.int32, L)
    av = [a_v[pl.ds(kk * L, L)] for kk in range(HG // L)]
    base0 = wid * EPW

    def prefetch(ch, b):
        pltpu.sync_copy(e2w_hbm.at[wid, ch], ebuf[b])
        pltpu.async_copy(h_hbm.at[ebuf[b].at[0]], hs_v[b], sem[b])
        pltpu.async_copy(h_hbm.at[ebuf[b].at[1]], hd_v[b], sem[b])

    def wait_gathers(b):
        pltpu.make_async_copy(h_hbm.at[ebuf[b].at[0]], hs_v[b], sem[b]).wait()
        pltpu.make_async_copy(h_hbm.at[ebuf[b].at[1]], hd_v[b], sem[b]).wait()

    prefetch(0, 0)

    def do_chunk(ch, b, first):
        wait_gathers(b)
        cp_ps = pltpu.async_copy(p_hbm.at[ebuf[b].at[0]], ps_v, sem_ps)
        if not first:
            pltpu.make_async_copy(pw_v, zsh.at[ebuf[1 - b].at[1]], sem_sc).wait()
        prefetch(ch + 1, 1 - b)

        exs = []
        for g in range(CHUNK // L):
            e_acc = jnp.zeros((L,), jnp.float32)
            q_acc = jnp.zeros((L,), jnp.float32)
            for r in range(L):
                row = g * L + r
                d0 = hs_v[b][row, pl.ds(0, L)] - hd_v[b][row, pl.ds(0, L)]
                d1 = hs_v[b][row, pl.ds(L, L)] - hd_v[b][row, pl.ds(L, L)]
                t = av[0] * jnp.abs(d0) + av[1] * jnp.abs(d1)
                q = d0 * d0 + d1 * d1
                e_r = jnp.sum(t)
                q_r = jnp.sum(q)
                e_acc = jnp.where(iota == r, e_r, e_acc)
                q_acc = jnp.where(iota == r, q_r, q_acc)
            ex = jnp.exp(jnp.maximum(e_acc, 0.0))
            exs.append(ex)
            exb_v[pl.ds(b * CHUNK + g * L, L)] = ex
            sqb_v[pl.ds(b * CHUNK + g * L, L)] = q_acc
        cp_ps.wait()

        for g in range(CHUNK // L):
            ex = exs[g]
            for r in range(L):
                row = g * L + r
                exi = ex[r]
                for j in range(D // L):
                    pw_v[row, pl.ds(j * L, L)] = ps_v[row, pl.ds(j * L, L)] * exi
                pw_v[row, pl.ds(D, L)] = jnp.where(iota == 0, exi, 0.0)

        pltpu.async_copy(pw_v, zsh.at[ebuf[b].at[1]], sem_sc, add=True)

    def flush_pair(pair):
        base = base0 + pair * 2 * CHUNK
        pltpu.sync_copy(exb_v, ex_hbm.at[pl.ds(base, 2 * CHUNK)])
        pltpu.sync_copy(sqb_v, sq_hbm.at[pl.ds(base, 2 * CHUNK)])

    do_chunk(0, 0, True)
    do_chunk(1, 1, False)
    flush_pair(0)

    def pair_body2(i, carry):
        do_chunk(2 * i, 0, False)
        do_chunk(2 * i + 1, 1, False)
        flush_pair(i)
        return carry

    lax.fori_loop(1, NCH // 2, pair_body2, 0)
    do_chunk(NCH - 1, 0, False)
    pltpu.sync_copy(exb_v.at[pl.ds(0, CHUNK)],
                    ex_hbm.at[pl.ds(base0 + (NCH - 1) * CHUNK, CHUNK)])
    pltpu.sync_copy(sqb_v.at[pl.ds(0, CHUNK)],
                    sq_hbm.at[pl.ds(base0 + (NCH - 1) * CHUNK, CHUNK)])
    pltpu.make_async_copy(pw_v, zsh.at[ebuf[0].at[1]], sem_sc).wait()
    wait_gathers(1)
    plsc.subcore_barrier()
    pltpu.sync_copy(zsh.at[pl.ds(r0, RPT)],
                    zacc_hbm.at[pl.ds(c * NP + r0, RPT)])


def _tc2_body(zacc_ref, b0_ref, w1_ref, q_ref, sinv_ref):
    za = zacc_ref[0] + zacc_ref[1]
    sden = za[:, D:D + 1]
    sinv = 1.0 / (sden + 1e-16)
    z = jnp.maximum(za[:, :D] * sinv + b0_ref[...], 0.0)
    q_ref[...] = jnp.dot(z, w1_ref[...], preferred_element_type=jnp.float32)
    sinv_ref[...] = sinv


def _tc2(zacc3, b0r, W1p):
    BR = 1000
    return pl.pallas_call(
        _tc2_body,
        grid=(N // BR,),
        in_specs=[pl.BlockSpec((2, BR, ZW), lambda i: (0, i, 0)),
                  pl.BlockSpec((1, D), lambda i: (0, 0)),
                  pl.BlockSpec((D, CP), lambda i: (0, 0))],
        out_specs=[pl.BlockSpec((BR, CP), lambda i: (i, 0)),
                   pl.BlockSpec((BR, 1), lambda i: (i, 0))],
        out_shape=[jax.ShapeDtypeStruct((N, CP), jnp.float32),
                   jax.ShapeDtypeStruct((N, 1), jnp.float32)],
    )(zacc3, b0r, W1p)


@functools.partial(
    pl.kernel,
    out_type=[
        jax.ShapeDtypeStruct((E,), jnp.float32),
        jax.ShapeDtypeStruct((NC * NP, CP), jnp.float32),
        jax.ShapeDtypeStruct((2 * NW, L), jnp.float32),
    ],
    mesh=_mesh,
    compiler_params=_sc_params,
    scratch_types=[
        pltpu.VMEM((N,), jnp.float32),
        pltpu.VMEM((NCH + 1, CHUNK), jnp.int32),
        pltpu.VMEM((NCH + 1, CHUNK), jnp.int32),
        pltpu.VMEM((EPW,), jnp.float32),
        pltpu.VMEM((EPW,), jnp.float32),
        pltpu.VMEM((EPW,), jnp.float32),
        [pltpu.VMEM((CHUNK, CP), jnp.float32)] * 2,
        pltpu.VMEM((CHUNK, CP), jnp.float32),
        pltpu.VMEM((2, L), jnp.float32),
        pltpu.VMEM_SHARED((NP, CP), jnp.float32),
        [pltpu.SemaphoreType.DMA] * 2,
        pltpu.SemaphoreType.DMA,
    ],
)
def _sc_pass3(q_hbm, sinv_hbm, srcw_hbm, dstw_hbm, exw_hbm, sqw_hbm, zero_hbm,
              w_hbm, oacc_hbm, lp_hbm,
              sinv_v, idxs_a, idxd_a, exa, sqa, wba, qs_v, qw_v, lbuf,
              osh, sem, sem_sc):
    c = lax.axis_index("c")
    s = lax.axis_index("s")
    wid = s * NC + c
    r0 = s * RPT
    pltpu.sync_copy(zero_hbm.at[pl.ds(r0, RPT)], osh.at[pl.ds(r0, RPT)])
    pltpu.sync_copy(sinv_hbm, sinv_v)
    pltpu.sync_copy(srcw_hbm.at[wid], idxs_a)
    pltpu.sync_copy(dstw_hbm.at[wid], idxd_a)
    pltpu.sync_copy(exw_hbm.at[wid], exa)
    pltpu.sync_copy(sqw_hbm.at[wid], sqa)
    plsc.subcore_barrier()

    def prefetch(ch, b):
        pltpu.async_copy(q_hbm.at[idxs_a.at[ch]], qs_v[b], sem[b])

    def wait_gathers(ch, b):
        pltpu.make_async_copy(q_hbm.at[idxs_a.at[ch]], qs_v[b], sem[b]).wait()

    prefetch(0, 0)

    def do_chunk(ch, b, carry, first=False):
        l1, l2 = carry
        wait_gathers(ch, b)
        if not first:
            pltpu.make_async_copy(qw_v, osh.at[idxd_a.at[ch - 1]],
                                  sem_sc).wait()
        prefetch(ch + 1, 1 - b)
        eoff = ch * CHUNK

        def group_body(g, car):
            a1, a2 = car
            esl = pl.ds(eoff + g * L, L)
            dvec = idxd_a[ch, pl.ds(g * L, L)]
            sv = plsc.load_gather(sinv_v, [dvec])
            wv = exa[esl] * sv
            wba[esl] = wv
            for r in range(L):
                row = g * L + r
                wi = wv[r]
                for j in range(CP // L):
                    qw_v[row, pl.ds(j * L, L)] = qs_v[b][row, pl.ds(j * L, L)] * wi
            return (a1 + wv * sqa[esl], a2 + wv * wv)

        l1, l2 = lax.fori_loop(0, CHUNK // L, group_body, (l1, l2))
        pltpu.async_copy(qw_v, osh.at[idxd_a.at[ch]], sem_sc, add=True)
        return (l1, l2)

    def pair_body(i, carry):
        carry = do_chunk(2 * i, 0, carry)
        carry = do_chunk(2 * i + 1, 1, carry)
        return carry

    zl = jnp.zeros((L,), jnp.float32)
    carry = do_chunk(0, 0, (zl, zl), first=True)
    carry = do_chunk(1, 1, carry)
    l1, l2 = lax.fori_loop(1, NCH // 2, pair_body, carry)
    l1, l2 = do_chunk(NCH - 1, 0, (l1, l2))
    pltpu.make_async_copy(qw_v, osh.at[idxd_a.at[NCH - 1]], sem_sc).wait()
    wait_gathers(NCH, 1)
    pltpu.sync_copy(wba, w_hbm.at[pl.ds(wid * EPW, EPW)])
    lbuf[0, :] = l1
    lbuf[1, :] = l2
    plsc.subcore_barrier()
    pltpu.sync_copy(osh.at[pl.ds(r0, RPT)],
                    oacc_hbm.at[pl.ds(c * NP + r0, RPT)])
    pltpu.sync_copy(lbuf, lp_hbm.at[pl.ds(wid * 2, 2)])


def _tc4_body(oacc_ref, b1_ref, lp_ref, z_ref, loss_ref):
    z_ref[...] = oacc_ref[0] + oacc_ref[1] + b1_ref[...]

    @pl.when(pl.program_id(0) == 0)
    def _():
        lp = lp_ref[...]
        rid = lax.broadcasted_iota(jnp.int32, lp.shape, 0)
        l1 = jnp.sum(jnp.where(rid % 2 == 0, lp, 0.0))
        l2 = jnp.sum(jnp.where(rid % 2 == 1, lp, 0.0))
        loss_ref[...] = jnp.reshape(LAMB1 * 0.5 * l1 + LAMB2 * l2, (1, 1))


def _tc4(oacc3, b1r, lp):
    BR = 1000
    return pl.pallas_call(
        _tc4_body,
        grid=(N // BR,),
        in_specs=[pl.BlockSpec((2, BR, CP), lambda i: (0, i, 0)),
                  pl.BlockSpec((1, CP), lambda i: (0, 0)),
                  pl.BlockSpec((2 * NW, L), lambda i: (0, 0))],
        out_specs=[pl.BlockSpec((BR, CP), lambda i: (i, 0)),
                   pl.BlockSpec((1, 1), lambda i: (0, 0))],
        out_shape=[jax.ShapeDtypeStruct((N, CP), jnp.float32),
                   jax.ShapeDtypeStruct((1, 1), jnp.float32)],
    )(oacc3, b1r, lp)


def kernel(x, edge_index, W_gl, a_gl, W0, b0, W1, b1):
    h, p = _tc1(x, W_gl, W0)

    zeros_z = jnp.zeros((NP, ZW), jnp.float32)
    srcw = jnp.pad(edge_index[0].reshape(NW, NCH, CHUNK),
                   ((0, 0), (0, 1), (0, 0)))
    dstw = jnp.pad(edge_index[1].reshape(NW, NCH, CHUNK),
                   ((0, 0), (0, 1), (0, 0)))
    e2w = jnp.stack([srcw, dstw], axis=2)
    zacc, ex, sq = _sc_pass1(h, p, e2w, a_gl, zeros_z)

    W1p = jnp.pad(W1, ((0, 0), (0, CP - C_OUT)))
    q, sinv = _tc2(zacc.reshape(NC, NP, ZW), b0.reshape(1, D), W1p)

    zeros_o = jnp.zeros((NP, CP), jnp.float32)
    exw = ex[:E].reshape(NW, EPW)
    sqw = sq[:E].reshape(NW, EPW)
    w, oacc, lp = _sc_pass3(q, sinv.reshape(N), srcw, dstw, exw, sqw, zeros_o)

    b1p = jnp.pad(b1, (0, CP - C_OUT)).reshape(1, CP)
    zout, loss = _tc4(oacc.reshape(NC, NP, CP), b1p, lp)
    return zout[:, :C_OUT], w, loss[0, 0]


# speedup vs baseline: 14.3679x; 1.1086x over previous
import functools

import jax
import jax.numpy as jnp
from jax import lax
from jax.experimental import pallas as pl
from jax.experimental.pallas import tpu as pltpu
from jax.experimental.pallas import tpu_sc as plsc

N = 10000
E = 320000
D = 128
HG = 32
H = 128
C_OUT = 40
CP = 48
ZW = 144
LAMB1 = 0.01
LAMB2 = 0.0001

NC = 2
NS = 16
NW = NC * NS
EPW = E // NW
CHUNK = 80
NCH = EPW // CHUNK
NP = 10112
RPT = NP // NS
L = 16

_mesh = plsc.VectorSubcoreMesh(core_axis_name="c", subcore_axis_name="s")
_sc_params = pltpu.CompilerParams(use_tc_tiling_on_sc=False,
                                  needs_layout_passes=False)


def _tc1_body(x_ref, wgl_ref, w0_ref, h_ref, p_ref):
    xb = x_ref[...]
    h_ref[...] = jnp.dot(xb, wgl_ref[...], preferred_element_type=jnp.float32)
    p_ref[...] = jnp.dot(xb, w0_ref[...], preferred_element_type=jnp.float32)


def _tc1(x, W_gl, W0):
    BR = 1000
    return pl.pallas_call(
        _tc1_body,
        grid=(N // BR,),
        in_specs=[pl.BlockSpec((BR, D), lambda i: (i, 0)),
                  pl.BlockSpec((D, HG), lambda i: (0, 0)),
                  pl.BlockSpec((D, H), lambda i: (0, 0))],
        out_specs=[pl.BlockSpec((BR, HG), lambda i: (i, 0)),
                   pl.BlockSpec((BR, H), lambda i: (i, 0))],
        out_shape=[jax.ShapeDtypeStruct((N, HG), jnp.float32),
                   jax.ShapeDtypeStruct((N, H), jnp.float32)],
    )(x, W_gl, W0)


@functools.partial(
    pl.kernel,
    out_type=[
        jax.ShapeDtypeStruct((NC * NP, ZW), jnp.float32),
        jax.ShapeDtypeStruct((E + CHUNK,), jnp.float32),
        jax.ShapeDtypeStruct((E + CHUNK,), jnp.float32),
    ],
    mesh=_mesh,
    compiler_params=_sc_params,
    scratch_types=[
        pltpu.VMEM((HG,), jnp.float32),
        [pltpu.VMEM((2, CHUNK), jnp.int32)] * 2,
        [pltpu.VMEM((CHUNK, HG), jnp.float32)] * 2,
        [pltpu.VMEM((CHUNK, HG), jnp.float32)] * 2,
        pltpu.VMEM((CHUNK, D), jnp.float32),
        pltpu.VMEM((CHUNK, ZW), jnp.float32),
        pltpu.VMEM((2 * CHUNK,), jnp.float32),
        pltpu.VMEM((2 * CHUNK,), jnp.float32),
        pltpu.VMEM_SHARED((NP, ZW), jnp.float32),
        [pltpu.SemaphoreType.DMA] * 2,
        pltpu.SemaphoreType.DMA,
        pltpu.SemaphoreType.DMA,
    ],
)
def _sc_pass1(h_hbm, p_hbm, e2w_hbm, a_hbm, zero_hbm,
              zacc_hbm, ex_hbm, sq_hbm,
              a_v, ebuf, hs_v, hd_v, ps_v, pw_v, exb_v, sqb_v,
              zsh, sem, sem_ps, sem_sc):
    c = lax.axis_index("c")
    s = lax.axis_index("s")
    wid = s * NC + c
    r0 = s * RPT
    pltpu.sync_copy(zero_hbm.at[pl.ds(r0, RPT)], zsh.at[pl.ds(r0, RPT)])
    pltpu.sync_copy(a_hbm, a_v)
    plsc.subcore_barrier()

    iota = lax.iota(jnp.int32, L)
    av = [a_v[pl.ds(kk * L, L)] for kk in range(HG // L)]
    base0 = wid * EPW

    def prefetch(ch, b):
        pltpu.sync_copy(e2w_hbm.at[wid, ch], ebuf[b])
        pltpu.async_copy(h_hbm.at[ebuf[b].at[0]], hs_v[b], sem[b])
        pltpu.async_copy(h_hbm.at[ebuf[b].at[1]], hd_v[b], sem[b])

    def wait_gathers(b):
        pltpu.make_async_copy(h_hbm.at[ebuf[b].at[0]], hs_v[b], sem[b]).wait()
        pltpu.make_async_copy(h_hbm.at[ebuf[b].at[1]], hd_v[b], sem[b]).wait()

    prefetch(0, 0)

    def do_chunk(ch, b, first):
        wait_gathers(b)
        cp_ps = pltpu.async_copy(p_hbm.at[ebuf[b].at[0]], ps_v, sem_ps)
        if not first:
            pltpu.make_async_copy(pw_v, zsh.at[ebuf[1 - b].at[1]], sem_sc).wait()
        prefetch(ch + 1, 1 - b)

        exs = []
        for g in range(CHUNK // L):
            e_acc = jnp.zeros((L,), jnp.float32)
            q_acc = jnp.zeros((L,), jnp.float32)
            for r in range(L):
                row = g * L + r
                d0 = hs_v[b][row, pl.ds(0, L)] - hd_v[b][row, pl.ds(0, L)]
                d1 = hs_v[b][row, pl.ds(L, L)] - hd_v[b][row, pl.ds(L, L)]
                t = av[0] * jnp.abs(d0) + av[1] * jnp.abs(d1)
                q = d0 * d0 + d1 * d1
                e_r = jnp.sum(t)
                q_r = jnp.sum(q)
                e_acc = jnp.where(iota == r, e_r, e_acc)
                q_acc = jnp.where(iota == r, q_r, q_acc)
            ex = jnp.exp(jnp.maximum(e_acc, 0.0))
            exs.append(ex)
            exb_v[pl.ds(b * CHUNK + g * L, L)] = ex
            sqb_v[pl.ds(b * CHUNK + g * L, L)] = q_acc
        cp_ps.wait()

        for g in range(CHUNK // L):
            ex = exs[g]
            for r in range(L):
                row = g * L + r
                exi = ex[r]
                for j in range(D // L):
                    pw_v[row, pl.ds(j * L, L)] = ps_v[row, pl.ds(j * L, L)] * exi
                pw_v[row, pl.ds(D, L)] = jnp.where(iota == 0, exi, 0.0)

        pltpu.async_copy(pw_v, zsh.at[ebuf[b].at[1]], sem_sc, add=True)

    def flush_pair(pair):
        base = base0 + pair * 2 * CHUNK
        pltpu.sync_copy(exb_v, ex_hbm.at[pl.ds(base, 2 * CHUNK)])
        pltpu.sync_copy(sqb_v, sq_hbm.at[pl.ds(base, 2 * CHUNK)])

    do_chunk(0, 0, True)
    do_chunk(1, 1, False)
    flush_pair(0)

    def pair_body2(i, carry):
        do_chunk(2 * i, 0, False)
        do_chunk(2 * i + 1, 1, False)
        flush_pair(i)
        return carry

    lax.fori_loop(1, NCH // 2, pair_body2, 0)
    do_chunk(NCH - 1, 0, False)
    pltpu.sync_copy(exb_v.at[pl.ds(0, CHUNK)],
                    ex_hbm.at[pl.ds(base0 + (NCH - 1) * CHUNK, CHUNK)])
    pltpu.sync_copy(sqb_v.at[pl.ds(0, CHUNK)],
                    sq_hbm.at[pl.ds(base0 + (NCH - 1) * CHUNK, CHUNK)])
    pltpu.make_async_copy(pw_v, zsh.at[ebuf[0].at[1]], sem_sc).wait()
    wait_gathers(1)
    plsc.subcore_barrier()
    pltpu.sync_copy(zsh.at[pl.ds(r0, RPT)],
                    zacc_hbm.at[pl.ds(c * NP + r0, RPT)])


def _tc2_body(zacc_ref, b0_ref, w1_ref, q_ref, sinv_ref):
    za = zacc_ref[0] + zacc_ref[1]
    sden = za[:, D:D + 1]
    sinv = 1.0 / (sden + 1e-16)
    z = jnp.maximum(za[:, :D] * sinv + b0_ref[...], 0.0)
    q_ref[...] = jnp.dot(z, w1_ref[...], preferred_element_type=jnp.float32)
    sinv_ref[...] = sinv


def _tc2(zacc3, b0r, W1p):
    BR = 1000
    return pl.pallas_call(
        _tc2_body,
        grid=(N // BR,),
        in_specs=[pl.BlockSpec((2, BR, ZW), lambda i: (0, i, 0)),
                  pl.BlockSpec((1, D), lambda i: (0, 0)),
                  pl.BlockSpec((D, CP), lambda i: (0, 0))],
        out_specs=[pl.BlockSpec((BR, CP), lambda i: (i, 0)),
                   pl.BlockSpec((BR, 1), lambda i: (i, 0))],
        out_shape=[jax.ShapeDtypeStruct((N, CP), jnp.float32),
                   jax.ShapeDtypeStruct((N, 1), jnp.float32)],
    )(zacc3, b0r, W1p)


@functools.partial(
    pl.kernel,
    out_type=[
        jax.ShapeDtypeStruct((E,), jnp.float32),
        jax.ShapeDtypeStruct((NC * NP, CP), jnp.float32),
        jax.ShapeDtypeStruct((2 * NW, L), jnp.float32),
    ],
    mesh=_mesh,
    compiler_params=_sc_params,
    scratch_types=[
        pltpu.VMEM((N,), jnp.float32),
        pltpu.VMEM((NCH + 1, CHUNK), jnp.int32),
        pltpu.VMEM((NCH + 1, CHUNK), jnp.int32),
        pltpu.VMEM((EPW,), jnp.float32),
        pltpu.VMEM((EPW,), jnp.float32),
        pltpu.VMEM((EPW,), jnp.float32),
        [pltpu.VMEM((CHUNK, CP), jnp.float32)] * 2,
        pltpu.VMEM((CHUNK, CP), jnp.float32),
        pltpu.VMEM((2, L), jnp.float32),
        pltpu.VMEM_SHARED((NP, CP), jnp.float32),
        [pltpu.SemaphoreType.DMA] * 2,
        pltpu.SemaphoreType.DMA,
    ],
)
def _sc_pass3(q_hbm, sinv_hbm, srcw_hbm, dstw_hbm, exw_hbm, sqw_hbm, zero_hbm,
              w_hbm, oacc_hbm, lp_hbm,
              sinv_v, idxs_a, idxd_a, exa, sqa, wba, qs_v, qw_v, lbuf,
              osh, sem, sem_sc):
    c = lax.axis_index("c")
    s = lax.axis_index("s")
    wid = s * NC + c
    r0 = s * RPT
    pltpu.sync_copy(zero_hbm.at[pl.ds(r0, RPT)], osh.at[pl.ds(r0, RPT)])
    pltpu.sync_copy(sinv_hbm, sinv_v)
    pltpu.sync_copy(srcw_hbm.at[wid], idxs_a)
    pltpu.sync_copy(dstw_hbm.at[wid], idxd_a)
    pltpu.sync_copy(exw_hbm.at[wid], exa)
    pltpu.sync_copy(sqw_hbm.at[wid], sqa)
    plsc.subcore_barrier()

    def prefetch(ch, b):
        pltpu.async_copy(q_hbm.at[idxs_a.at[ch]], qs_v[b], sem[b])

    def wait_gathers(ch, b):
        pltpu.make_async_copy(q_hbm.at[idxs_a.at[ch]], qs_v[b], sem[b]).wait()

    prefetch(0, 0)

    def do_chunk(ch, b, carry, first=False):
        l1, l2 = carry
        wait_gathers(ch, b)
        if not first:
            pltpu.make_async_copy(qw_v, osh.at[idxd_a.at[ch - 1]],
                                  sem_sc).wait()
        prefetch(ch + 1, 1 - b)
        eoff = ch * CHUNK

        for g in range(CHUNK // L):
            esl = pl.ds(eoff + g * L, L)
            dvec = idxd_a[ch, pl.ds(g * L, L)]
            sv = plsc.load_gather(sinv_v, [dvec])
            wv = exa[esl] * sv
            wba[esl] = wv
            for r in range(L):
                row = g * L + r
                wi = wv[r]
                for j in range(CP // L):
                    qw_v[row, pl.ds(j * L, L)] = qs_v[b][row, pl.ds(j * L, L)] * wi
            l1 = l1 + wv * sqa[esl]
            l2 = l2 + wv * wv
        pltpu.async_copy(qw_v, osh.at[idxd_a.at[ch]], sem_sc, add=True)
        return (l1, l2)

    def pair_body(i, carry):
        carry = do_chunk(2 * i, 0, carry)
        carry = do_chunk(2 * i + 1, 1, carry)
        return carry

    zl = jnp.zeros((L,), jnp.float32)
    carry = do_chunk(0, 0, (zl, zl), first=True)
    carry = do_chunk(1, 1, carry)
    l1, l2 = lax.fori_loop(1, NCH // 2, pair_body, carry)
    l1, l2 = do_chunk(NCH - 1, 0, (l1, l2))
    pltpu.make_async_copy(qw_v, osh.at[idxd_a.at[NCH - 1]], sem_sc).wait()
    wait_gathers(NCH, 1)
    pltpu.sync_copy(wba, w_hbm.at[pl.ds(wid * EPW, EPW)])
    lbuf[0, :] = l1
    lbuf[1, :] = l2
    plsc.subcore_barrier()
    pltpu.sync_copy(osh.at[pl.ds(r0, RPT)],
                    oacc_hbm.at[pl.ds(c * NP + r0, RPT)])
    pltpu.sync_copy(lbuf, lp_hbm.at[pl.ds(wid * 2, 2)])


def _tc4_body(oacc_ref, b1_ref, lp_ref, z_ref, loss_ref):
    z_ref[...] = oacc_ref[0] + oacc_ref[1] + b1_ref[...]

    @pl.when(pl.program_id(0) == 0)
    def _():
        lp = lp_ref[...]
        rid = lax.broadcasted_iota(jnp.int32, lp.shape, 0)
        l1 = jnp.sum(jnp.where(rid % 2 == 0, lp, 0.0))
        l2 = jnp.sum(jnp.where(rid % 2 == 1, lp, 0.0))
        loss_ref[...] = jnp.reshape(LAMB1 * 0.5 * l1 + LAMB2 * l2, (1, 1))


def _tc4(oacc3, b1r, lp):
    BR = 1000
    return pl.pallas_call(
        _tc4_body,
        grid=(N // BR,),
        in_specs=[pl.BlockSpec((2, BR, CP), lambda i: (0, i, 0)),
                  pl.BlockSpec((1, CP), lambda i: (0, 0)),
                  pl.BlockSpec((2 * NW, L), lambda i: (0, 0))],
        out_specs=[pl.BlockSpec((BR, CP), lambda i: (i, 0)),
                   pl.BlockSpec((1, 1), lambda i: (0, 0))],
        out_shape=[jax.ShapeDtypeStruct((N, CP), jnp.float32),
                   jax.ShapeDtypeStruct((1, 1), jnp.float32)],
    )(oacc3, b1r, lp)


def kernel(x, edge_index, W_gl, a_gl, W0, b0, W1, b1):
    h, p = _tc1(x, W_gl, W0)

    zeros_z = jnp.zeros((NP, ZW), jnp.float32)
    srcw = jnp.pad(edge_index[0].reshape(NW, NCH, CHUNK),
                   ((0, 0), (0, 1), (0, 0)))
    dstw = jnp.pad(edge_index[1].reshape(NW, NCH, CHUNK),
                   ((0, 0), (0, 1), (0, 0)))
    e2w = jnp.stack([srcw, dstw], axis=2)
    zacc, ex, sq = _sc_pass1(h, p, e2w, a_gl, zeros_z)

    W1p = jnp.pad(W1, ((0, 0), (0, CP - C_OUT)))
    q, sinv = _tc2(zacc.reshape(NC, NP, ZW), b0.reshape(1, D), W1p)

    zeros_o = jnp.zeros((NP, CP), jnp.float32)
    exw = ex[:E].reshape(NW, EPW)
    sqw = sq[:E].reshape(NW, EPW)
    w, oacc, lp = _sc_pass3(q, sinv.reshape(N), srcw, dstw, exw, sqw, zeros_o)

    b1p = jnp.pad(b1, (0, CP - C_OUT)).reshape(1, CP)
    zout, loss = _tc4(oacc.reshape(NC, NP, CP), b1p, lp)
    return zout[:, :C_OUT], w, loss[0, 0]
